# Initial kernel scaffold; baseline (speedup 1.0000x reference)
import jax
import jax.numpy as jnp
from jax.experimental import pallas as pl


def kernel(ufeat, ifeat, W_user, W_item, ufc_W, ufc_b, ifc_W, ifc_b, P, combine_W, enc_src, enc_dst, enc_etype, dec_u, dec_i):
    raise NotImplementedError("write your pallas kernel here")


# speedup vs baseline: 5.2442x; 5.2442x over previous
import jax
import jax.numpy as jnp
from jax import lax
from jax.experimental import pallas as pl
from jax.experimental.pallas import tpu as pltpu
from jax.experimental.pallas import tpu_sc as plsc

NU = 5000
NI = 5000
E_ENC = 320000
E_DEC = 320000
D_IN = 128
AGG = 128
OUT = 64
R = 5
NBASIS = 2

NCORE = 2
NSUB = 16
LANES = 16

NP = 5120
RPT = NP // NSUB

CH_A = 2000
EA_T = E_ENC // NSUB

SUB = 128
NSC = 3
CH_C = SUB * NSC
EC_T = 20352
EPAD_C = NSUB * EC_T

ED_T = 10368
EPAD_D = 32 * ED_T

_f32 = jnp.float32
_i32 = jnp.int32


def _deg_body(src_hbm, dst_hbm, degu_hbm, degi_hbm, ebuf, hist, red):
    cid = lax.axis_index("c")
    sid = lax.axis_index("s")
    lane = lax.broadcasted_iota(_i32, (LANES,), 0)
    ones = jnp.ones((LANES,), _f32)
    zero16 = jnp.zeros((LANES,), _f32)

    def zf(i, c):
        hist[pl.ds(i * LANES, LANES)] = zero16
        return c

    lax.fori_loop(0, (NSUB * NP) // LANES, zf, 0)

    def _process(edge_hbm):
        def ck(k, c):
            base = sid * EA_T + k * CH_A
            pltpu.sync_copy(edge_hbm.at[pl.ds(base, CH_A)], ebuf)

            def gp(g, c2):
                v = ebuf[pl.ds(g * LANES, LANES)]
                plsc.addupdate_scatter(hist, [lane * NP + v], ones)
                return c2

            lax.fori_loop(0, CH_A // LANES, gp, 0)
            return c

        lax.fori_loop(0, EA_T // CH_A, ck, 0)

        def rd(j, c):
            acc = zero16
            for l in range(LANES):
                acc = acc + hist[pl.ds(l * NP + j * LANES, LANES)]
            red[0, pl.ds(j * LANES, LANES)] = acc
            return c

        lax.fori_loop(0, NP // LANES, rd, 0)

    @pl.when(cid == 0)
    def _():
        _process(src_hbm)
        pltpu.sync_copy(red, degu_hbm.at[sid])

    @pl.when(cid == 1)
    def _():
        _process(dst_hbm)
        pltpu.sync_copy(red, degi_hbm.at[sid])


_deg_call = pl.kernel(
    _deg_body,
    out_type=[jax.ShapeDtypeStruct((NSUB, 1, NP), _f32),
              jax.ShapeDtypeStruct((NSUB, 1, NP), _f32)],
    mesh=plsc.VectorSubcoreMesh(core_axis_name="c", subcore_axis_name="s"),
    scratch_types=[pltpu.VMEM((CH_A,), _i32),
                   pltpu.VMEM((NSUB * NP,), _f32),
                   pltpu.VMEM((1, NP), _f32)],
    compiler_params=pltpu.CompilerParams(needs_layout_passes=False),
)


BN = 1000


def _tables_body(uf, vf, wu, wi, du, di, tu, ti):
    cu = lax.rsqrt(jnp.maximum(jnp.sum(du[...], axis=0)[:NU], 1.0))
    ci = lax.rsqrt(jnp.maximum(jnp.sum(di[...], axis=0)[:NI], 1.0))
    su = uf[...] * cu[:, None]
    si = vf[...] * ci[:, None]
    tu[...] = jnp.dot(su, wu[0], preferred_element_type=_f32)[None]
    ti[...] = jnp.dot(si, wi[0], preferred_element_type=_f32)[None]


def _tables_call(ufeat, ifeat, W_user, W_item, degu, degi):
    return pl.pallas_call(
        _tables_body,
        grid=(R,),
        in_specs=[
            pl.BlockSpec((NU, D_IN), lambda r: (0, 0)),
            pl.BlockSpec((NI, D_IN), lambda r: (0, 0)),
            pl.BlockSpec((1, D_IN, AGG), lambda r: (r, 0, 0)),
            pl.BlockSpec((1, D_IN, AGG), lambda r: (r, 0, 0)),
            pl.BlockSpec((NSUB, NP), lambda r: (0, 0)),
            pl.BlockSpec((NSUB, NP), lambda r: (0, 0)),
        ],
        out_specs=[
            pl.BlockSpec((1, NU, AGG), lambda r: (r, 0, 0)),
            pl.BlockSpec((1, NI, AGG), lambda r: (r, 0, 0)),
        ],
        out_shape=[jax.ShapeDtypeStruct((R, NU, AGG), _f32),
                   jax.ShapeDtypeStruct((R, NI, AGG), _f32)],
    )(ufeat, ifeat, W_user, W_item, degu, degi)


def _enc_body(tu_hbm, ti_hbm, src_hbm, dst_hbm, et_hbm, hi_hbm, hu_hbm,
              sbuf, dbuf, ebuf, kg, ks, rows, zbuf, accum, sem):
    cid = lax.axis_index("c")
    sid = lax.axis_index("s")
    zero16 = jnp.zeros((LANES,), _f32)

    def zf(i, c):
        for j in range(SUB // LANES):
            zbuf[i, pl.ds(j * LANES, LANES)] = zero16
        return c

    lax.fori_loop(0, SUB, zf, 0)
    r0 = sid * RPT
    pltpu.sync_copy(zbuf, accum.at[pl.ds(r0, SUB)])
    pltpu.sync_copy(zbuf, accum.at[pl.ds(r0 + SUB, SUB)])
    pltpu.sync_copy(zbuf.at[pl.ds(0, RPT - 2 * SUB)],
                    accum.at[pl.ds(r0 + 2 * SUB, RPT - 2 * SUB)])
    plsc.subcore_barrier()

    def _run(table_hbm, g_hbm, s_hbm):
        def ck(k, c):
            base = sid * EC_T + k * CH_C
            pltpu.sync_copy(g_hbm.at[pl.ds(base, CH_C)], sbuf)
            pltpu.sync_copy(s_hbm.at[pl.ds(base, CH_C)], dbuf)
            pltpu.sync_copy(et_hbm.at[pl.ds(base, CH_C)], ebuf)
            for j in range(NSC):
                def gp(i, c2, j=j):
                    o = j * SUB + i * LANES
                    et = ebuf[pl.ds(o, LANES)]
                    gv = sbuf[pl.ds(o, LANES)]
                    sv = dbuf[pl.ds(o, LANES)]
                    kg[j, pl.ds(i * LANES, LANES)] = et * NU + gv
                    ks[j, pl.ds(i * LANES, LANES)] = sv
                    return c2
                lax.fori_loop(0, SUB // LANES, gp, 0)
            cps = [pltpu.async_copy(table_hbm.at[kg.at[j]], rows.at[j], sem)
                   for j in range(NSC)]
            for cp in cps:
                cp.wait()
            for j in range(NSC):
                pltpu.sync_copy(rows.at[j], accum.at[ks.at[j]], add=True)
            return c

        lax.fori_loop(0, EC_T // CH_C, ck, 0)

    @pl.when(cid == 0)
    def _():
        _run(tu_hbm, src_hbm, dst_hbm)

    @pl.when(cid == 1)
    def _():
        _run(ti_hbm, dst_hbm, src_hbm)

    plsc.subcore_barrier()

    @pl.when(cid == 0)
    def _():
        pltpu.sync_copy(accum.at[pl.ds(r0, RPT)], hi_hbm.at[pl.ds(r0, RPT)])

    @pl.when(cid == 1)
    def _():
        pltpu.sync_copy(accum.at[pl.ds(r0, RPT)], hu_hbm.at[pl.ds(r0, RPT)])


_enc_call = pl.kernel(
    _enc_body,
    out_type=[jax.ShapeDtypeStruct((NP, AGG), _f32),
              jax.ShapeDtypeStruct((NP, AGG), _f32)],
    mesh=plsc.VectorSubcoreMesh(core_axis_name="c", subcore_axis_name="s"),
    scratch_types=[pltpu.VMEM((CH_C,), _i32),
                   pltpu.VMEM((CH_C,), _i32),
                   pltpu.VMEM((CH_C,), _i32),
                   pltpu.VMEM((NSC, SUB), _i32),
                   pltpu.VMEM((NSC, SUB), _i32),
                   pltpu.VMEM((NSC, SUB, AGG), _f32),
                   pltpu.VMEM((SUB, AGG), _f32),
                   pltpu.VMEM_SHARED((NP, AGG), _f32),
                   pltpu.SemaphoreType.DMA],
    compiler_params=pltpu.CompilerParams(needs_layout_passes=False),
)


def _fc_body(hi, hu, du, di, ufw, ufb, ifw, ifb, p, dtu, dti):
    cu = lax.rsqrt(jnp.maximum(jnp.sum(du[...], axis=0)[:NU], 1.0))
    ci = lax.rsqrt(jnp.maximum(jnp.sum(di[...], axis=0)[:NI], 1.0))
    au = hu[...] * cu[:, None]
    ai = hi[...] * ci[:, None]
    au = jnp.where(au >= 0, au, 0.1 * au)
    ai = jnp.where(ai >= 0, ai, 0.1 * ai)
    uo = jnp.dot(au, ufw[...], preferred_element_type=_f32) + ufb[...]
    io = jnp.dot(ai, ifw[...], preferred_element_type=_f32) + ifb[...]
    dtu[...] = jnp.concatenate(
        [jnp.dot(uo, p[0], preferred_element_type=_f32),
         jnp.dot(uo, p[1], preferred_element_type=_f32)], axis=1)
    dti[...] = jnp.concatenate([io, jnp.zeros((NI, OUT), _f32)], axis=1)


def _fc_call(hi, hu, degu, degi, ufc_W, ufc_b, ifc_W, ifc_b, P):
    return pl.pallas_call(
        _fc_body,
        grid=(1,),
        in_specs=[
            pl.BlockSpec((NU, AGG), lambda n: (0, 0)),
            pl.BlockSpec((NU, AGG), lambda n: (0, 0)),
            pl.BlockSpec((NSUB, NP), lambda n: (0, 0)),
            pl.BlockSpec((NSUB, NP), lambda n: (0, 0)),
            pl.BlockSpec((AGG, OUT), lambda n: (0, 0)),
            pl.BlockSpec((1, OUT), lambda n: (0, 0)),
            pl.BlockSpec((AGG, OUT), lambda n: (0, 0)),
            pl.BlockSpec((1, OUT), lambda n: (0, 0)),
            pl.BlockSpec((NBASIS, OUT, OUT), lambda n: (0, 0, 0)),
        ],
        out_specs=[
            pl.BlockSpec((NU, 2 * OUT), lambda n: (0, 0)),
            pl.BlockSpec((NU, 2 * OUT), lambda n: (0, 0)),
        ],
        out_shape=[jax.ShapeDtypeStruct((NU, 2 * OUT), _f32),
                   jax.ShapeDtypeStruct((NI, 2 * OUT), _f32)],
    )(hi, hu, degu, degi, ufc_W, ufc_b, ifc_W, ifc_b, P)


def _dec_body(dtu_hbm, dti_hbm, du_hbm, di_hbm, b0_hbm, b1_hbm,
              iu, ii, urows, irows, b0b, b1b, sem):
    cid = lax.axis_index("c")
    sid = lax.axis_index("s")
    wid = sid * NCORE + cid
    lane = lax.broadcasted_iota(_i32, (LANES,), 0)
    zero16 = jnp.zeros((LANES,), _f32)

    def ck(k, c):
        base = wid * ED_T + k * CH_C
        for j in range(NSC):
            pltpu.sync_copy(du_hbm.at[pl.ds(base + j * SUB, SUB)], iu.at[j])
            pltpu.sync_copy(di_hbm.at[pl.ds(base + j * SUB, SUB)], ii.at[j])
        cps = [pltpu.async_copy(dtu_hbm.at[iu.at[j]], urows.at[j], sem)
               for j in range(NSC)]
        cps += [pltpu.async_copy(dti_hbm.at[ii.at[j]], irows.at[j], sem)
                for j in range(NSC)]
        for cp in cps:
            cp.wait()
        for j in range(NSC):
            def gp(g, c2, j=j):
                ridx = lane + g * LANES
                acc0 = zero16
                acc1 = zero16
                for t in range(OUT):
                    ct = jnp.full((LANES,), t, _i32)
                    ct2 = jnp.full((LANES,), t + OUT, _i32)
                    icol = plsc.load_gather(irows.at[j], [ridx, ct])
                    u0 = plsc.load_gather(urows.at[j], [ridx, ct])
                    u1 = plsc.load_gather(urows.at[j], [ridx, ct2])
                    acc0 = acc0 + u0 * icol
                    acc1 = acc1 + u1 * icol
                b0b[pl.ds(j * SUB + g * LANES, LANES)] = acc0
                b1b[pl.ds(j * SUB + g * LANES, LANES)] = acc1
                return c2
            lax.fori_loop(0, SUB // LANES, gp, 0)
        pltpu.sync_copy(b0b, b0_hbm.at[pl.ds(base, CH_C)])
        pltpu.sync_copy(b1b, b1_hbm.at[pl.ds(base, CH_C)])
        return c

    lax.fori_loop(0, ED_T // CH_C, ck, 0)


_dec_call = pl.kernel(
    _dec_body,
    out_type=[jax.ShapeDtypeStruct((EPAD_D,), _f32),
              jax.ShapeDtypeStruct((EPAD_D,), _f32)],
    mesh=plsc.VectorSubcoreMesh(core_axis_name="c", subcore_axis_name="s"),
    scratch_types=[pltpu.VMEM((NSC, SUB), _i32),
                   pltpu.VMEM((NSC, SUB), _i32),
                   pltpu.VMEM((NSC, SUB, 2 * OUT), _f32),
                   pltpu.VMEM((NSC, SUB, 2 * OUT), _f32),
                   pltpu.VMEM((CH_C,), _f32),
                   pltpu.VMEM((CH_C,), _f32),
                   pltpu.SemaphoreType.DMA],
    compiler_params=pltpu.CompilerParams(needs_layout_passes=False),
)


BF = 3200


def _comb_body(b0, b1, cw, out):
    out[...] = b0[...] * cw[0:1, :] + b1[...] * cw[1:2, :]


def _comb_call(b0, b1, combine_W):
    return pl.pallas_call(
        _comb_body,
        grid=(E_DEC // BF,),
        in_specs=[
            pl.BlockSpec((BF, 1), lambda n: (n, 0)),
            pl.BlockSpec((BF, 1), lambda n: (n, 0)),
            pl.BlockSpec((NBASIS, R), lambda n: (0, 0)),
        ],
        out_specs=pl.BlockSpec((BF, R), lambda n: (n, 0)),
        out_shape=jax.ShapeDtypeStruct((E_DEC, R), _f32),
    )(b0, b1, combine_W)


def kernel(ufeat, ifeat, W_user, W_item, ufc_W, ufc_b, ifc_W, ifc_b, P,
           combine_W, enc_src, enc_dst, enc_etype, dec_u, dec_i):
    enc_src = enc_src.astype(_i32)
    enc_dst = enc_dst.astype(_i32)
    enc_etype = enc_etype.astype(_i32)
    dec_u = dec_u.astype(_i32)
    dec_i = dec_i.astype(_i32)

    degu, degi = _deg_call(enc_src, enc_dst)
    degu = degu.reshape(NSUB, NP)
    degi = degi.reshape(NSUB, NP)
    tu, ti = _tables_call(ufeat, ifeat, W_user, W_item, degu, degi)
    tu = tu.reshape(R * NU, AGG)
    ti = ti.reshape(R * NI, AGG)

    pc = EPAD_C - E_ENC
    src_p = jnp.concatenate([enc_src, jnp.full((pc,), NU, _i32)])
    dst_p = jnp.concatenate([enc_dst, jnp.full((pc,), NU, _i32)])
    et_p = jnp.concatenate([enc_etype, jnp.zeros((pc,), _i32)])
    hi_raw, hu_raw = _enc_call(tu, ti, src_p, dst_p, et_p)

    dtu, dti = _fc_call(hi_raw[:NU], hu_raw[:NU], degu, degi,
                        ufc_W, ufc_b.reshape(1, OUT),
                        ifc_W, ifc_b.reshape(1, OUT), P)

    pd = EPAD_D - E_DEC
    du_p = jnp.concatenate([dec_u, jnp.zeros((pd,), _i32)])
    di_p = jnp.concatenate([dec_i, jnp.zeros((pd,), _i32)])
    b0, b1 = _dec_call(dtu, dti, du_p, di_p)

    return _comb_call(b0[:E_DEC].reshape(E_DEC, 1),
                      b1[:E_DEC].reshape(E_DEC, 1), combine_W)


# speedup vs baseline: 6.8697x; 1.3100x over previous
import jax
import jax.numpy as jnp
from jax import lax
from jax.experimental import pallas as pl
from jax.experimental.pallas import tpu as pltpu
from jax.experimental.pallas import tpu_sc as plsc

NU = 5000
NI = 5000
E_ENC = 320000
E_DEC = 320000
D_IN = 128
AGG = 128
OUT = 64
R = 5
NBASIS = 2

NCORE = 2
NSUB = 16
LANES = 16

NP = 5120
RPT = NP // NSUB

CH_A = 2000
EA_T = E_ENC // NSUB

SUB = 128
NSC = 3
CH_C = SUB * NSC
EC_T = 20352
EPAD_C = NSUB * EC_T

ED_T = 10368
EPAD_D = 32 * ED_T

_f32 = jnp.float32
_i32 = jnp.int32


def _deg_body(src_hbm, dst_hbm, degu_hbm, degi_hbm, ebuf, hist, red):
    cid = lax.axis_index("c")
    sid = lax.axis_index("s")
    lane = lax.broadcasted_iota(_i32, (LANES,), 0)
    ones = jnp.ones((LANES,), _f32)
    zero16 = jnp.zeros((LANES,), _f32)

    def zf(i, c):
        hist[pl.ds(i * LANES, LANES)] = zero16
        return c

    lax.fori_loop(0, (NSUB * NP) // LANES, zf, 0)

    def _process(edge_hbm):
        def ck(k, c):
            base = sid * EA_T + k * CH_A
            pltpu.sync_copy(edge_hbm.at[pl.ds(base, CH_A)], ebuf)

            def gp(g, c2):
                v = ebuf[pl.ds(g * LANES, LANES)]
                plsc.addupdate_scatter(hist, [lane * NP + v], ones)
                return c2

            lax.fori_loop(0, CH_A // LANES, gp, 0)
            return c

        lax.fori_loop(0, EA_T // CH_A, ck, 0)

        def rd(j, c):
            acc = zero16
            for l in range(LANES):
                acc = acc + hist[pl.ds(l * NP + j * LANES, LANES)]
            red[0, pl.ds(j * LANES, LANES)] = acc
            return c

        lax.fori_loop(0, NP // LANES, rd, 0)

    @pl.when(cid == 0)
    def _():
        _process(src_hbm)
        pltpu.sync_copy(red, degu_hbm.at[sid])

    @pl.when(cid == 1)
    def _():
        _process(dst_hbm)
        pltpu.sync_copy(red, degi_hbm.at[sid])


_deg_call = pl.kernel(
    _deg_body,
    out_type=[jax.ShapeDtypeStruct((NSUB, 1, NP), _f32),
              jax.ShapeDtypeStruct((NSUB, 1, NP), _f32)],
    mesh=plsc.VectorSubcoreMesh(core_axis_name="c", subcore_axis_name="s"),
    scratch_types=[pltpu.VMEM((CH_A,), _i32),
                   pltpu.VMEM((NSUB * NP,), _f32),
                   pltpu.VMEM((1, NP), _f32)],
    compiler_params=pltpu.CompilerParams(needs_layout_passes=False),
)


BN = 1000


def _tables_body(uf, vf, wu, wi, du, di, tu, ti):
    cu = lax.rsqrt(jnp.maximum(jnp.sum(du[...], axis=0)[:NU], 1.0))
    ci = lax.rsqrt(jnp.maximum(jnp.sum(di[...], axis=0)[:NI], 1.0))
    su = uf[...] * cu[:, None]
    si = vf[...] * ci[:, None]
    tu[...] = jnp.dot(su, wu[0], preferred_element_type=_f32)[None]
    ti[...] = jnp.dot(si, wi[0], preferred_element_type=_f32)[None]


def _tables_call(ufeat, ifeat, W_user, W_item, degu, degi):
    return pl.pallas_call(
        _tables_body,
        grid=(R,),
        in_specs=[
            pl.BlockSpec((NU, D_IN), lambda r: (0, 0)),
            pl.BlockSpec((NI, D_IN), lambda r: (0, 0)),
            pl.BlockSpec((1, D_IN, AGG), lambda r: (r, 0, 0)),
            pl.BlockSpec((1, D_IN, AGG), lambda r: (r, 0, 0)),
            pl.BlockSpec((NSUB, NP), lambda r: (0, 0)),
            pl.BlockSpec((NSUB, NP), lambda r: (0, 0)),
        ],
        out_specs=[
            pl.BlockSpec((1, NU, AGG), lambda r: (r, 0, 0)),
            pl.BlockSpec((1, NI, AGG), lambda r: (r, 0, 0)),
        ],
        out_shape=[jax.ShapeDtypeStruct((R, NU, AGG), _f32),
                   jax.ShapeDtypeStruct((R, NI, AGG), _f32)],
    )(ufeat, ifeat, W_user, W_item, degu, degi)


def _enc_body(tu_hbm, ti_hbm, src_hbm, dst_hbm, et_hbm, hi_hbm, hu_hbm,
              sbuf, dbuf, ebuf, kg, ks, rows, zbuf, accum, sem):
    cid = lax.axis_index("c")
    sid = lax.axis_index("s")
    zero16 = jnp.zeros((LANES,), _f32)

    def zf(i, c):
        for j in range(SUB // LANES):
            zbuf[i, pl.ds(j * LANES, LANES)] = zero16
        return c

    lax.fori_loop(0, SUB, zf, 0)
    r0 = sid * RPT
    pltpu.sync_copy(zbuf, accum.at[pl.ds(r0, SUB)])
    pltpu.sync_copy(zbuf, accum.at[pl.ds(r0 + SUB, SUB)])
    pltpu.sync_copy(zbuf.at[pl.ds(0, RPT - 2 * SUB)],
                    accum.at[pl.ds(r0 + 2 * SUB, RPT - 2 * SUB)])
    plsc.subcore_barrier()

    def _run(table_hbm, g_hbm, s_hbm):
        def ck(k, c):
            base = sid * EC_T + k * CH_C
            pltpu.sync_copy(g_hbm.at[pl.ds(base, CH_C)], sbuf)
            pltpu.sync_copy(s_hbm.at[pl.ds(base, CH_C)], dbuf)
            pltpu.sync_copy(et_hbm.at[pl.ds(base, CH_C)], ebuf)
            for j in range(NSC):
                def gp(i, c2, j=j):
                    o = j * SUB + i * LANES
                    et = ebuf[pl.ds(o, LANES)]
                    gv = sbuf[pl.ds(o, LANES)]
                    sv = dbuf[pl.ds(o, LANES)]
                    kg[j, pl.ds(i * LANES, LANES)] = et * NU + gv
                    ks[j, pl.ds(i * LANES, LANES)] = sv
                    return c2
                lax.fori_loop(0, SUB // LANES, gp, 0)
            cps = [pltpu.async_copy(table_hbm.at[kg.at[j]], rows.at[j], sem)
                   for j in range(NSC)]
            for cp in cps:
                cp.wait()
            for j in range(NSC):
                pltpu.sync_copy(rows.at[j], accum.at[ks.at[j]], add=True)
            return c

        lax.fori_loop(0, EC_T // CH_C, ck, 0)

    @pl.when(cid == 0)
    def _():
        _run(tu_hbm, src_hbm, dst_hbm)

    @pl.when(cid == 1)
    def _():
        _run(ti_hbm, dst_hbm, src_hbm)

    plsc.subcore_barrier()

    @pl.when(cid == 0)
    def _():
        pltpu.sync_copy(accum.at[pl.ds(r0, RPT)], hi_hbm.at[pl.ds(r0, RPT)])

    @pl.when(cid == 1)
    def _():
        pltpu.sync_copy(accum.at[pl.ds(r0, RPT)], hu_hbm.at[pl.ds(r0, RPT)])


_enc_call = pl.kernel(
    _enc_body,
    out_type=[jax.ShapeDtypeStruct((NP, AGG), _f32),
              jax.ShapeDtypeStruct((NP, AGG), _f32)],
    mesh=plsc.VectorSubcoreMesh(core_axis_name="c", subcore_axis_name="s"),
    scratch_types=[pltpu.VMEM((CH_C,), _i32),
                   pltpu.VMEM((CH_C,), _i32),
                   pltpu.VMEM((CH_C,), _i32),
                   pltpu.VMEM((NSC, SUB), _i32),
                   pltpu.VMEM((NSC, SUB), _i32),
                   pltpu.VMEM((NSC, SUB, AGG), _f32),
                   pltpu.VMEM((SUB, AGG), _f32),
                   pltpu.VMEM_SHARED((NP, AGG), _f32),
                   pltpu.SemaphoreType.DMA],
    compiler_params=pltpu.CompilerParams(needs_layout_passes=False),
)


def _fc_body(hi, hu, du, di, ufw, ufb, ifw, ifb, p, dtu, dti):
    cu = lax.rsqrt(jnp.maximum(jnp.sum(du[...], axis=0)[:NU], 1.0))
    ci = lax.rsqrt(jnp.maximum(jnp.sum(di[...], axis=0)[:NI], 1.0))
    au = hu[...] * cu[:, None]
    ai = hi[...] * ci[:, None]
    au = jnp.where(au >= 0, au, 0.1 * au)
    ai = jnp.where(ai >= 0, ai, 0.1 * ai)
    uo = jnp.dot(au, ufw[...], preferred_element_type=_f32) + ufb[...]
    io = jnp.dot(ai, ifw[...], preferred_element_type=_f32) + ifb[...]
    dtu[...] = jnp.concatenate(
        [jnp.dot(uo, p[0], preferred_element_type=_f32),
         jnp.dot(uo, p[1], preferred_element_type=_f32)], axis=1)
    dti[...] = jnp.concatenate([io, jnp.zeros((NI, OUT), _f32)], axis=1)


def _fc_call(hi, hu, degu, degi, ufc_W, ufc_b, ifc_W, ifc_b, P):
    return pl.pallas_call(
        _fc_body,
        grid=(1,),
        in_specs=[
            pl.BlockSpec((NU, AGG), lambda n: (0, 0)),
            pl.BlockSpec((NU, AGG), lambda n: (0, 0)),
            pl.BlockSpec((NSUB, NP), lambda n: (0, 0)),
            pl.BlockSpec((NSUB, NP), lambda n: (0, 0)),
            pl.BlockSpec((AGG, OUT), lambda n: (0, 0)),
            pl.BlockSpec((1, OUT), lambda n: (0, 0)),
            pl.BlockSpec((AGG, OUT), lambda n: (0, 0)),
            pl.BlockSpec((1, OUT), lambda n: (0, 0)),
            pl.BlockSpec((NBASIS, OUT, OUT), lambda n: (0, 0, 0)),
        ],
        out_specs=[
            pl.BlockSpec((NU, 2 * OUT), lambda n: (0, 0)),
            pl.BlockSpec((NU, 2 * OUT), lambda n: (0, 0)),
        ],
        out_shape=[jax.ShapeDtypeStruct((NU, 2 * OUT), _f32),
                   jax.ShapeDtypeStruct((NI, 2 * OUT), _f32)],
    )(hi, hu, degu, degi, ufc_W, ufc_b, ifc_W, ifc_b, P)


def _dec_body(dtu_hbm, dti_hbm, du_hbm, di_hbm, b0_hbm, b1_hbm,
              iu, ii, urows, irows, b0b, b1b, sem):
    cid = lax.axis_index("c")
    sid = lax.axis_index("s")
    wid = sid * NCORE + cid
    lane = lax.broadcasted_iota(_i32, (LANES,), 0)
    zero16 = jnp.zeros((LANES,), _f32)

    def ck(k, c):
        base = wid * ED_T + k * CH_C
        for j in range(NSC):
            pltpu.sync_copy(du_hbm.at[pl.ds(base + j * SUB, SUB)], iu.at[j])
            pltpu.sync_copy(di_hbm.at[pl.ds(base + j * SUB, SUB)], ii.at[j])
        cps = [pltpu.async_copy(dtu_hbm.at[iu.at[j]], urows.at[j], sem)
               for j in range(NSC)]
        cps += [pltpu.async_copy(dti_hbm.at[ii.at[j]], irows.at[j], sem)
                for j in range(NSC)]
        for cp in cps:
            cp.wait()
        for j in range(NSC):
            def gp(g, c2, j=j):
                e0 = g * LANES
                b0v = zero16
                b1v = zero16
                for i in range(LANES):
                    e = e0 + i
                    uv = [urows[j, e, pl.ds(t * LANES, LANES)]
                          for t in range(8)]
                    iv = [irows[j, e, pl.ds(t * LANES, LANES)]
                          for t in range(4)]
                    p0 = uv[0] * iv[0] + uv[1] * iv[1]                         + uv[2] * iv[2] + uv[3] * iv[3]
                    p1 = uv[4] * iv[0] + uv[5] * iv[1]                         + uv[6] * iv[2] + uv[7] * iv[3]
                    b0v = jnp.where(lane == i, jnp.sum(p0), b0v)
                    b1v = jnp.where(lane == i, jnp.sum(p1), b1v)
                b0b[pl.ds(j * SUB + e0, LANES)] = b0v
                b1b[pl.ds(j * SUB + e0, LANES)] = b1v
                return c2
            lax.fori_loop(0, SUB // LANES, gp, 0)
        pltpu.sync_copy(b0b, b0_hbm.at[pl.ds(base, CH_C)])
        pltpu.sync_copy(b1b, b1_hbm.at[pl.ds(base, CH_C)])
        return c

    lax.fori_loop(0, ED_T // CH_C, ck, 0)


_dec_call = pl.kernel(
    _dec_body,
    out_type=[jax.ShapeDtypeStruct((EPAD_D,), _f32),
              jax.ShapeDtypeStruct((EPAD_D,), _f32)],
    mesh=plsc.VectorSubcoreMesh(core_axis_name="c", subcore_axis_name="s"),
    scratch_types=[pltpu.VMEM((NSC, SUB), _i32),
                   pltpu.VMEM((NSC, SUB), _i32),
                   pltpu.VMEM((NSC, SUB, 2 * OUT), _f32),
                   pltpu.VMEM((NSC, SUB, 2 * OUT), _f32),
                   pltpu.VMEM((CH_C,), _f32),
                   pltpu.VMEM((CH_C,), _f32),
                   pltpu.SemaphoreType.DMA],
    compiler_params=pltpu.CompilerParams(needs_layout_passes=False),
)


BF = 3200


def _comb_body(b0, b1, cw, out):
    out[...] = b0[...] * cw[0:1, :] + b1[...] * cw[1:2, :]


def _comb_call(b0, b1, combine_W):
    return pl.pallas_call(
        _comb_body,
        grid=(E_DEC // BF,),
        in_specs=[
            pl.BlockSpec((BF, 1), lambda n: (n, 0)),
            pl.BlockSpec((BF, 1), lambda n: (n, 0)),
            pl.BlockSpec((NBASIS, R), lambda n: (0, 0)),
        ],
        out_specs=pl.BlockSpec((BF, R), lambda n: (n, 0)),
        out_shape=jax.ShapeDtypeStruct((E_DEC, R), _f32),
    )(b0, b1, combine_W)


def kernel(ufeat, ifeat, W_user, W_item, ufc_W, ufc_b, ifc_W, ifc_b, P,
           combine_W, enc_src, enc_dst, enc_etype, dec_u, dec_i):
    enc_src = enc_src.astype(_i32)
    enc_dst = enc_dst.astype(_i32)
    enc_etype = enc_etype.astype(_i32)
    dec_u = dec_u.astype(_i32)
    dec_i = dec_i.astype(_i32)

    degu, degi = _deg_call(enc_src, enc_dst)
    degu = degu.reshape(NSUB, NP)
    degi = degi.reshape(NSUB, NP)
    tu, ti = _tables_call(ufeat, ifeat, W_user, W_item, degu, degi)
    tu = tu.reshape(R * NU, AGG)
    ti = ti.reshape(R * NI, AGG)

    pc = EPAD_C - E_ENC
    src_p = jnp.concatenate([enc_src, jnp.full((pc,), NU, _i32)])
    dst_p = jnp.concatenate([enc_dst, jnp.full((pc,), NU, _i32)])
    et_p = jnp.concatenate([enc_etype, jnp.zeros((pc,), _i32)])
    hi_raw, hu_raw = _enc_call(tu, ti, src_p, dst_p, et_p)

    dtu, dti = _fc_call(hi_raw[:NU], hu_raw[:NU], degu, degi,
                        ufc_W, ufc_b.reshape(1, OUT),
                        ifc_W, ifc_b.reshape(1, OUT), P)

    pd = EPAD_D - E_DEC
    du_p = jnp.concatenate([dec_u, jnp.zeros((pd,), _i32)])
    di_p = jnp.concatenate([dec_i, jnp.zeros((pd,), _i32)])
    b0, b1 = _dec_call(dtu, dti, du_p, di_p)

    return _comb_call(b0[:E_DEC].reshape(E_DEC, 1),
                      b1[:E_DEC].reshape(E_DEC, 1), combine_W)


# speedup vs baseline: 6.8823x; 1.0018x over previous
import jax
import jax.numpy as jnp
from jax import lax
from jax.experimental import pallas as pl
from jax.experimental.pallas import tpu as pltpu
from jax.experimental.pallas import tpu_sc as plsc

NU = 5000
NI = 5000
E_ENC = 320000
E_DEC = 320000
D_IN = 128
AGG = 128
OUT = 64
R = 5
NBASIS = 2

NCORE = 2
NSUB = 16
LANES = 16

NP = 5120
RPT = NP // NSUB

CH_A = 2000
EA_T = E_ENC // NSUB

SUB = 128
NSC = 3
CH_C = SUB * NSC
NSC_C = 5
CH_CC = SUB * NSC_C
EC_T = 20480
EPAD_C = NSUB * EC_T
NCH_C = EC_T // CH_CC

ED_T = 10368
EPAD_D = 32 * ED_T

_f32 = jnp.float32
_i32 = jnp.int32


def _deg_body(src_hbm, dst_hbm, degu_hbm, degi_hbm, ebuf, hist, red):
    cid = lax.axis_index("c")
    sid = lax.axis_index("s")
    lane = lax.broadcasted_iota(_i32, (LANES,), 0)
    ones = jnp.ones((LANES,), _f32)
    zero16 = jnp.zeros((LANES,), _f32)

    def zf(i, c):
        hist[pl.ds(i * LANES, LANES)] = zero16
        return c

    lax.fori_loop(0, (NSUB * NP) // LANES, zf, 0)

    def _process(edge_hbm):
        def ck(k, c):
            base = sid * EA_T + k * CH_A
            pltpu.sync_copy(edge_hbm.at[pl.ds(base, CH_A)], ebuf)

            def gp(g, c2):
                v = ebuf[pl.ds(g * LANES, LANES)]
                plsc.addupdate_scatter(hist, [lane * NP + v], ones)
                return c2

            lax.fori_loop(0, CH_A // LANES, gp, 0)
            return c

        lax.fori_loop(0, EA_T // CH_A, ck, 0)

        def rd(j, c):
            acc = zero16
            for l in range(LANES):
                acc = acc + hist[pl.ds(l * NP + j * LANES, LANES)]
            red[0, pl.ds(j * LANES, LANES)] = acc
            return c

        lax.fori_loop(0, NP // LANES, rd, 0)

    @pl.when(cid == 0)
    def _():
        _process(src_hbm)
        pltpu.sync_copy(red, degu_hbm.at[sid])

    @pl.when(cid == 1)
    def _():
        _process(dst_hbm)
        pltpu.sync_copy(red, degi_hbm.at[sid])


_deg_call = pl.kernel(
    _deg_body,
    out_type=[jax.ShapeDtypeStruct((NSUB, 1, NP), _f32),
              jax.ShapeDtypeStruct((NSUB, 1, NP), _f32)],
    mesh=plsc.VectorSubcoreMesh(core_axis_name="c", subcore_axis_name="s"),
    scratch_types=[pltpu.VMEM((CH_A,), _i32),
                   pltpu.VMEM((NSUB * NP,), _f32),
                   pltpu.VMEM((1, NP), _f32)],
    compiler_params=pltpu.CompilerParams(needs_layout_passes=False),
)


BN = 1000


def _tables_body(uf, vf, wu, wi, du, di, tu, ti):
    cu = lax.rsqrt(jnp.maximum(jnp.sum(du[...], axis=0)[:NU], 1.0))
    ci = lax.rsqrt(jnp.maximum(jnp.sum(di[...], axis=0)[:NI], 1.0))
    su = uf[...] * cu[:, None]
    si = vf[...] * ci[:, None]
    tu[...] = jnp.dot(su, wu[0], preferred_element_type=_f32)[None]
    ti[...] = jnp.dot(si, wi[0], preferred_element_type=_f32)[None]


def _tables_call(ufeat, ifeat, W_user, W_item, degu, degi):
    return pl.pallas_call(
        _tables_body,
        grid=(R,),
        in_specs=[
            pl.BlockSpec((NU, D_IN), lambda r: (0, 0)),
            pl.BlockSpec((NI, D_IN), lambda r: (0, 0)),
            pl.BlockSpec((1, D_IN, AGG), lambda r: (r, 0, 0)),
            pl.BlockSpec((1, D_IN, AGG), lambda r: (r, 0, 0)),
            pl.BlockSpec((NSUB, NP), lambda r: (0, 0)),
            pl.BlockSpec((NSUB, NP), lambda r: (0, 0)),
        ],
        out_specs=[
            pl.BlockSpec((1, NU, AGG), lambda r: (r, 0, 0)),
            pl.BlockSpec((1, NI, AGG), lambda r: (r, 0, 0)),
        ],
        out_shape=[jax.ShapeDtypeStruct((R, NU, AGG), _f32),
                   jax.ShapeDtypeStruct((R, NI, AGG), _f32)],
    )(ufeat, ifeat, W_user, W_item, degu, degi)


def _enc_body(tu_hbm, ti_hbm, encx_hbm, hi_hbm, hu_hbm,
              ibuf, kg, ks, rows, accum, sem, ssem):
    cid = lax.axis_index("c")
    sid = lax.axis_index("s")
    zero16 = jnp.zeros((LANES,), _f32)

    def zf(i, c):
        for j in range(3):
            for t in range(SUB // LANES):
                rows[j, i, pl.ds(t * LANES, LANES)] = zero16
        return c

    lax.fori_loop(0, SUB, zf, 0)
    r0 = sid * RPT
    pltpu.sync_copy(rows.at[0], accum.at[pl.ds(r0, SUB)])
    pltpu.sync_copy(rows.at[1], accum.at[pl.ds(r0 + SUB, SUB)])
    pltpu.sync_copy(rows.at[2].at[pl.ds(0, RPT - 2 * SUB)],
                    accum.at[pl.ds(r0 + 2 * SUB, RPT - 2 * SUB)])
    plsc.subcore_barrier()

    def _run(table_hbm, g_row, s_row):
        def ck(k, c):
            pltpu.sync_copy(encx_hbm.at[sid * NCH_C + k], ibuf)
            for j in range(NSC_C):
                def gp(i, c2, j=j):
                    o = j * SUB + i * LANES
                    et = ibuf[2, pl.ds(o, LANES)]
                    gv = ibuf[g_row, pl.ds(o, LANES)]
                    sv = ibuf[s_row, pl.ds(o, LANES)]
                    kg[j, pl.ds(i * LANES, LANES)] = et * NU + gv
                    ks[j, pl.ds(i * LANES, LANES)] = sv
                    return c2
                lax.fori_loop(0, SUB // LANES, gp, 0)
            cps = [pltpu.async_copy(table_hbm.at[kg.at[j]], rows.at[j], sem)
                   for j in range(NSC_C)]
            for cp in cps:
                cp.wait()
            cps2 = [pltpu.async_copy(rows.at[j], accum.at[ks.at[j]], ssem,
                                     add=True)
                    for j in range(NSC_C)]
            for cp in cps2:
                cp.wait()
            return c

        lax.fori_loop(0, NCH_C, ck, 0)

    @pl.when(cid == 0)
    def _():
        _run(tu_hbm, 0, 1)

    @pl.when(cid == 1)
    def _():
        _run(ti_hbm, 1, 0)

    plsc.subcore_barrier()

    @pl.when(cid == 0)
    def _():
        pltpu.sync_copy(accum.at[pl.ds(r0, RPT)], hi_hbm.at[pl.ds(r0, RPT)])

    @pl.when(cid == 1)
    def _():
        pltpu.sync_copy(accum.at[pl.ds(r0, RPT)], hu_hbm.at[pl.ds(r0, RPT)])


_enc_call = pl.kernel(
    _enc_body,
    out_type=[jax.ShapeDtypeStruct((NP, AGG), _f32),
              jax.ShapeDtypeStruct((NP, AGG), _f32)],
    mesh=plsc.VectorSubcoreMesh(core_axis_name="c", subcore_axis_name="s"),
    scratch_types=[pltpu.VMEM((3, CH_CC), _i32),
                   pltpu.VMEM((NSC_C, SUB), _i32),
                   pltpu.VMEM((NSC_C, SUB), _i32),
                   pltpu.VMEM((NSC_C, SUB, AGG), _f32),
                   pltpu.VMEM_SHARED((NP, AGG), _f32),
                   pltpu.SemaphoreType.DMA,
                   pltpu.SemaphoreType.DMA],
    compiler_params=pltpu.CompilerParams(needs_layout_passes=False),
)


def _fc_body(hi, hu, du, di, ufw, ufb, ifw, ifb, p, dtu, dti):
    cu = lax.rsqrt(jnp.maximum(jnp.sum(du[...], axis=0)[:NU], 1.0))
    ci = lax.rsqrt(jnp.maximum(jnp.sum(di[...], axis=0)[:NI], 1.0))
    au = hu[...] * cu[:, None]
    ai = hi[...] * ci[:, None]
    au = jnp.where(au >= 0, au, 0.1 * au)
    ai = jnp.where(ai >= 0, ai, 0.1 * ai)
    uo = jnp.dot(au, ufw[...], preferred_element_type=_f32) + ufb[...]
    io = jnp.dot(ai, ifw[...], preferred_element_type=_f32) + ifb[...]
    dtu[...] = jnp.concatenate(
        [jnp.dot(uo, p[0], preferred_element_type=_f32),
         jnp.dot(uo, p[1], preferred_element_type=_f32)], axis=1)
    dti[...] = jnp.concatenate([io, jnp.zeros((NI, OUT), _f32)], axis=1)


def _fc_call(hi, hu, degu, degi, ufc_W, ufc_b, ifc_W, ifc_b, P):
    return pl.pallas_call(
        _fc_body,
        grid=(1,),
        in_specs=[
            pl.BlockSpec((NU, AGG), lambda n: (0, 0)),
            pl.BlockSpec((NU, AGG), lambda n: (0, 0)),
            pl.BlockSpec((NSUB, NP), lambda n: (0, 0)),
            pl.BlockSpec((NSUB, NP), lambda n: (0, 0)),
            pl.BlockSpec((AGG, OUT), lambda n: (0, 0)),
            pl.BlockSpec((1, OUT), lambda n: (0, 0)),
            pl.BlockSpec((AGG, OUT), lambda n: (0, 0)),
            pl.BlockSpec((1, OUT), lambda n: (0, 0)),
            pl.BlockSpec((NBASIS, OUT, OUT), lambda n: (0, 0, 0)),
        ],
        out_specs=[
            pl.BlockSpec((NU, 2 * OUT), lambda n: (0, 0)),
            pl.BlockSpec((NU, 2 * OUT), lambda n: (0, 0)),
        ],
        out_shape=[jax.ShapeDtypeStruct((NU, 2 * OUT), _f32),
                   jax.ShapeDtypeStruct((NI, 2 * OUT), _f32)],
    )(hi, hu, degu, degi, ufc_W, ufc_b, ifc_W, ifc_b, P)


def _dec_body(dtu_hbm, dti_hbm, du_hbm, di_hbm, b0_hbm, b1_hbm,
              iu, ii, urows, irows, b0b, b1b, pbuf0, pbuf1, sem):
    cid = lax.axis_index("c")
    sid = lax.axis_index("s")
    wid = sid * NCORE + cid
    lane = lax.broadcasted_iota(_i32, (LANES,), 0)
    zero16 = jnp.zeros((LANES,), _f32)

    def ck(k, c):
        base = wid * ED_T + k * CH_C
        for j in range(NSC):
            pltpu.sync_copy(du_hbm.at[pl.ds(base + j * SUB, SUB)], iu.at[j])
            pltpu.sync_copy(di_hbm.at[pl.ds(base + j * SUB, SUB)], ii.at[j])
        cps = [pltpu.async_copy(dtu_hbm.at[iu.at[j]], urows.at[j], sem)
               for j in range(NSC)]
        cps += [pltpu.async_copy(dti_hbm.at[ii.at[j]], irows.at[j], sem)
                for j in range(NSC)]
        for cp in cps:
            cp.wait()
        for j in range(NSC):
            def gp(g, c2, j=j):
                e0 = g * LANES
                for i in range(LANES):
                    e = e0 + i
                    uv = [urows[j, e, pl.ds(t * LANES, LANES)]
                          for t in range(8)]
                    iv = [irows[j, e, pl.ds(t * LANES, LANES)]
                          for t in range(4)]
                    p0 = uv[0] * iv[0] + uv[1] * iv[1]                         + uv[2] * iv[2] + uv[3] * iv[3]
                    p1 = uv[4] * iv[0] + uv[5] * iv[1]                         + uv[6] * iv[2] + uv[7] * iv[3]
                    pbuf0[i, pl.ds(0, LANES)] = p0
                    pbuf1[i, pl.ds(0, LANES)] = p1
                b0v = zero16
                b1v = zero16
                for t in range(LANES):
                    cc = jnp.full((LANES,), t, _i32)
                    b0v = b0v + plsc.load_gather(pbuf0, [lane, cc])
                    b1v = b1v + plsc.load_gather(pbuf1, [lane, cc])
                b0b[pl.ds(j * SUB + e0, LANES)] = b0v
                b1b[pl.ds(j * SUB + e0, LANES)] = b1v
                return c2
            lax.fori_loop(0, SUB // LANES, gp, 0)
        pltpu.sync_copy(b0b, b0_hbm.at[pl.ds(base, CH_C)])
        pltpu.sync_copy(b1b, b1_hbm.at[pl.ds(base, CH_C)])
        return c

    lax.fori_loop(0, ED_T // CH_C, ck, 0)


_dec_call = pl.kernel(
    _dec_body,
    out_type=[jax.ShapeDtypeStruct((EPAD_D,), _f32),
              jax.ShapeDtypeStruct((EPAD_D,), _f32)],
    mesh=plsc.VectorSubcoreMesh(core_axis_name="c", subcore_axis_name="s"),
    scratch_types=[pltpu.VMEM((NSC, SUB), _i32),
                   pltpu.VMEM((NSC, SUB), _i32),
                   pltpu.VMEM((NSC, SUB, 2 * OUT), _f32),
                   pltpu.VMEM((NSC, SUB, 2 * OUT), _f32),
                   pltpu.VMEM((CH_C,), _f32),
                   pltpu.VMEM((CH_C,), _f32),
                   pltpu.VMEM((LANES, LANES + 1), _f32),
                   pltpu.VMEM((LANES, LANES + 1), _f32),
                   pltpu.SemaphoreType.DMA],
    compiler_params=pltpu.CompilerParams(needs_layout_passes=False),
)


BF = 3200


def _comb_body(b0, b1, cw, out):
    out[...] = b0[...] * cw[0:1, :] + b1[...] * cw[1:2, :]


def _comb_call(b0, b1, combine_W):
    return pl.pallas_call(
        _comb_body,
        grid=(E_DEC // BF,),
        in_specs=[
            pl.BlockSpec((BF, 1), lambda n: (n, 0)),
            pl.BlockSpec((BF, 1), lambda n: (n, 0)),
            pl.BlockSpec((NBASIS, R), lambda n: (0, 0)),
        ],
        out_specs=pl.BlockSpec((BF, R), lambda n: (n, 0)),
        out_shape=jax.ShapeDtypeStruct((E_DEC, R), _f32),
    )(b0, b1, combine_W)


def kernel(ufeat, ifeat, W_user, W_item, ufc_W, ufc_b, ifc_W, ifc_b, P,
           combine_W, enc_src, enc_dst, enc_etype, dec_u, dec_i):
    enc_src = enc_src.astype(_i32)
    enc_dst = enc_dst.astype(_i32)
    enc_etype = enc_etype.astype(_i32)
    dec_u = dec_u.astype(_i32)
    dec_i = dec_i.astype(_i32)

    degu, degi = _deg_call(enc_src, enc_dst)
    degu = degu.reshape(NSUB, NP)
    degi = degi.reshape(NSUB, NP)
    tu, ti = _tables_call(ufeat, ifeat, W_user, W_item, degu, degi)
    tu = tu.reshape(R * NU, AGG)
    ti = ti.reshape(R * NI, AGG)

    pc = EPAD_C - E_ENC
    src_p = jnp.concatenate([enc_src, jnp.full((pc,), NU, _i32)])
    dst_p = jnp.concatenate([enc_dst, jnp.full((pc,), NU, _i32)])
    et_p = jnp.concatenate([enc_etype, jnp.zeros((pc,), _i32)])
    encx = jnp.stack([src_p.reshape(-1, CH_CC), dst_p.reshape(-1, CH_CC),
                      et_p.reshape(-1, CH_CC)], axis=1)
    hi_raw, hu_raw = _enc_call(tu, ti, encx)

    dtu, dti = _fc_call(hi_raw[:NU], hu_raw[:NU], degu, degi,
                        ufc_W, ufc_b.reshape(1, OUT),
                        ifc_W, ifc_b.reshape(1, OUT), P)

    pd = EPAD_D - E_DEC
    du_p = jnp.concatenate([dec_u, jnp.zeros((pd,), _i32)])
    di_p = jnp.concatenate([dec_i, jnp.zeros((pd,), _i32)])
    b0, b1 = _dec_call(dtu, dti, du_p, di_p)

    return _comb_call(b0[:E_DEC].reshape(E_DEC, 1),
                      b1[:E_DEC].reshape(E_DEC, 1), combine_W)


# speedup vs baseline: 9.0862x; 1.3202x over previous
import jax
import jax.numpy as jnp
from jax import lax
from jax.experimental import pallas as pl
from jax.experimental.pallas import tpu as pltpu
from jax.experimental.pallas import tpu_sc as plsc

NU = 5000
NI = 5000
E_ENC = 320000
E_DEC = 320000
D_IN = 128
AGG = 128
OUT = 64
R = 5
NBASIS = 2

NCORE = 2
NSUB = 16
LANES = 16

NP = 5120
RPT = NP // NSUB

CH_A = 2000
EA_T = E_ENC // NSUB

SUB = 128
NSC = 3
CH_C = SUB * NSC
NSC_C = 5
CH_CC = SUB * NSC_C
EC_T = 20480
EPAD_C = NSUB * EC_T
NCH_C = EC_T // CH_CC

ED_T = 10368
EPAD_D = 32 * ED_T
NCH_E = ED_T // SUB
NBUF_E = 3
NGRP_E = NCH_E // NBUF_E

_f32 = jnp.float32
_i32 = jnp.int32


def _deg_body(src_hbm, dst_hbm, degu_hbm, degi_hbm, ebuf, hist, red):
    cid = lax.axis_index("c")
    sid = lax.axis_index("s")
    lane = lax.broadcasted_iota(_i32, (LANES,), 0)
    ones = jnp.ones((LANES,), _f32)
    zero16 = jnp.zeros((LANES,), _f32)

    def zf(i, c):
        hist[pl.ds(i * LANES, LANES)] = zero16
        return c

    lax.fori_loop(0, (NSUB * NP) // LANES, zf, 0)

    def _process(edge_hbm):
        def ck(k, c):
            base = sid * EA_T + k * CH_A
            pltpu.sync_copy(edge_hbm.at[pl.ds(base, CH_A)], ebuf)

            def gp(g, c2):
                v = ebuf[pl.ds(g * LANES, LANES)]
                plsc.addupdate_scatter(hist, [lane * NP + v], ones)
                return c2

            lax.fori_loop(0, CH_A // LANES, gp, 0)
            return c

        lax.fori_loop(0, EA_T // CH_A, ck, 0)

        def rd(j, c):
            acc = zero16
            for l in range(LANES):
                acc = acc + hist[pl.ds(l * NP + j * LANES, LANES)]
            red[0, pl.ds(j * LANES, LANES)] = acc
            return c

        lax.fori_loop(0, NP // LANES, rd, 0)

    @pl.when(cid == 0)
    def _():
        _process(src_hbm)
        pltpu.sync_copy(red, degu_hbm.at[sid])

    @pl.when(cid == 1)
    def _():
        _process(dst_hbm)
        pltpu.sync_copy(red, degi_hbm.at[sid])


_deg_call = pl.kernel(
    _deg_body,
    out_type=[jax.ShapeDtypeStruct((NSUB, 1, NP), _f32),
              jax.ShapeDtypeStruct((NSUB, 1, NP), _f32)],
    mesh=plsc.VectorSubcoreMesh(core_axis_name="c", subcore_axis_name="s"),
    scratch_types=[pltpu.VMEM((CH_A,), _i32),
                   pltpu.VMEM((NSUB * NP,), _f32),
                   pltpu.VMEM((1, NP), _f32)],
    compiler_params=pltpu.CompilerParams(needs_layout_passes=False),
)


BN = 1000


def _tables_body(uf, vf, wu, wi, du, di, tu, ti):
    cu = lax.rsqrt(jnp.maximum(jnp.sum(du[...], axis=0)[:NU], 1.0))
    ci = lax.rsqrt(jnp.maximum(jnp.sum(di[...], axis=0)[:NI], 1.0))
    su = uf[...] * cu[:, None]
    si = vf[...] * ci[:, None]
    tu[...] = jnp.dot(su, wu[0], preferred_element_type=_f32)[None]
    ti[...] = jnp.dot(si, wi[0], preferred_element_type=_f32)[None]


def _tables_call(ufeat, ifeat, W_user, W_item, degu, degi):
    return pl.pallas_call(
        _tables_body,
        grid=(R,),
        in_specs=[
            pl.BlockSpec((NU, D_IN), lambda r: (0, 0)),
            pl.BlockSpec((NI, D_IN), lambda r: (0, 0)),
            pl.BlockSpec((1, D_IN, AGG), lambda r: (r, 0, 0)),
            pl.BlockSpec((1, D_IN, AGG), lambda r: (r, 0, 0)),
            pl.BlockSpec((NSUB, NP), lambda r: (0, 0)),
            pl.BlockSpec((NSUB, NP), lambda r: (0, 0)),
        ],
        out_specs=[
            pl.BlockSpec((1, NU, AGG), lambda r: (r, 0, 0)),
            pl.BlockSpec((1, NI, AGG), lambda r: (r, 0, 0)),
        ],
        out_shape=[jax.ShapeDtypeStruct((R, NU, AGG), _f32),
                   jax.ShapeDtypeStruct((R, NI, AGG), _f32)],
    )(ufeat, ifeat, W_user, W_item, degu, degi)


def _enc_body(tu_hbm, ti_hbm, encx_hbm, hi_hbm, hu_hbm,
              ibuf, kg, ks, rows, accum, sem, ssem):
    cid = lax.axis_index("c")
    sid = lax.axis_index("s")
    zero16 = jnp.zeros((LANES,), _f32)

    def zf(i, c):
        for j in range(3):
            for t in range(SUB // LANES):
                rows[j, i, pl.ds(t * LANES, LANES)] = zero16
        return c

    lax.fori_loop(0, SUB, zf, 0)
    r0 = sid * RPT
    pltpu.sync_copy(rows.at[0], accum.at[pl.ds(r0, SUB)])
    pltpu.sync_copy(rows.at[1], accum.at[pl.ds(r0 + SUB, SUB)])
    pltpu.sync_copy(rows.at[2].at[pl.ds(0, RPT - 2 * SUB)],
                    accum.at[pl.ds(r0 + 2 * SUB, RPT - 2 * SUB)])
    plsc.subcore_barrier()

    def _run(table_hbm, g_row, s_row):
        def ck(k, c):
            pltpu.sync_copy(encx_hbm.at[sid * NCH_C + k], ibuf)
            for j in range(NSC_C):
                def gp(i, c2, j=j):
                    o = j * SUB + i * LANES
                    et = ibuf[2, pl.ds(o, LANES)]
                    gv = ibuf[g_row, pl.ds(o, LANES)]
                    sv = ibuf[s_row, pl.ds(o, LANES)]
                    kg[j, pl.ds(i * LANES, LANES)] = et * NU + gv
                    ks[j, pl.ds(i * LANES, LANES)] = sv
                    return c2
                lax.fori_loop(0, SUB // LANES, gp, 0)
            cps = [pltpu.async_copy(table_hbm.at[kg.at[j]], rows.at[j], sem)
                   for j in range(NSC_C)]
            for cp in cps:
                cp.wait()
            cps2 = [pltpu.async_copy(rows.at[j], accum.at[ks.at[j]], ssem,
                                     add=True)
                    for j in range(NSC_C)]
            for cp in cps2:
                cp.wait()
            return c

        lax.fori_loop(0, NCH_C, ck, 0)

    @pl.when(cid == 0)
    def _():
        _run(tu_hbm, 0, 1)

    @pl.when(cid == 1)
    def _():
        _run(ti_hbm, 1, 0)

    plsc.subcore_barrier()

    @pl.when(cid == 0)
    def _():
        pltpu.sync_copy(accum.at[pl.ds(r0, RPT)], hi_hbm.at[pl.ds(r0, RPT)])

    @pl.when(cid == 1)
    def _():
        pltpu.sync_copy(accum.at[pl.ds(r0, RPT)], hu_hbm.at[pl.ds(r0, RPT)])


_enc_call = pl.kernel(
    _enc_body,
    out_type=[jax.ShapeDtypeStruct((NP, AGG), _f32),
              jax.ShapeDtypeStruct((NP, AGG), _f32)],
    mesh=plsc.VectorSubcoreMesh(core_axis_name="c", subcore_axis_name="s"),
    scratch_types=[pltpu.VMEM((3, CH_CC), _i32),
                   pltpu.VMEM((NSC_C, SUB), _i32),
                   pltpu.VMEM((NSC_C, SUB), _i32),
                   pltpu.VMEM((NSC_C, SUB, AGG), _f32),
                   pltpu.VMEM_SHARED((NP, AGG), _f32),
                   pltpu.SemaphoreType.DMA,
                   pltpu.SemaphoreType.DMA],
    compiler_params=pltpu.CompilerParams(needs_layout_passes=False),
)


def _fc_body(hi, hu, du, di, ufw, ufb, ifw, ifb, p, dtu, dti):
    cu = lax.rsqrt(jnp.maximum(jnp.sum(du[...], axis=0)[:NU], 1.0))
    ci = lax.rsqrt(jnp.maximum(jnp.sum(di[...], axis=0)[:NI], 1.0))
    au = hu[...] * cu[:, None]
    ai = hi[...] * ci[:, None]
    au = jnp.where(au >= 0, au, 0.1 * au)
    ai = jnp.where(ai >= 0, ai, 0.1 * ai)
    uo = jnp.dot(au, ufw[...], preferred_element_type=_f32) + ufb[...]
    io = jnp.dot(ai, ifw[...], preferred_element_type=_f32) + ifb[...]
    dtu[...] = jnp.concatenate(
        [jnp.dot(uo, p[0], preferred_element_type=_f32),
         jnp.dot(uo, p[1], preferred_element_type=_f32)], axis=1)
    dti[...] = jnp.concatenate([io, jnp.zeros((NI, OUT), _f32)], axis=1)


def _fc_call(hi, hu, degu, degi, ufc_W, ufc_b, ifc_W, ifc_b, P):
    return pl.pallas_call(
        _fc_body,
        grid=(1,),
        in_specs=[
            pl.BlockSpec((NU, AGG), lambda n: (0, 0)),
            pl.BlockSpec((NU, AGG), lambda n: (0, 0)),
            pl.BlockSpec((NSUB, NP), lambda n: (0, 0)),
            pl.BlockSpec((NSUB, NP), lambda n: (0, 0)),
            pl.BlockSpec((AGG, OUT), lambda n: (0, 0)),
            pl.BlockSpec((1, OUT), lambda n: (0, 0)),
            pl.BlockSpec((AGG, OUT), lambda n: (0, 0)),
            pl.BlockSpec((1, OUT), lambda n: (0, 0)),
            pl.BlockSpec((NBASIS, OUT, OUT), lambda n: (0, 0, 0)),
        ],
        out_specs=[
            pl.BlockSpec((NU, 2 * OUT), lambda n: (0, 0)),
            pl.BlockSpec((NU, 2 * OUT), lambda n: (0, 0)),
        ],
        out_shape=[jax.ShapeDtypeStruct((NU, 2 * OUT), _f32),
                   jax.ShapeDtypeStruct((NI, 2 * OUT), _f32)],
    )(hi, hu, degu, degi, ufc_W, ufc_b, ifc_W, ifc_b, P)


def _dec_body(dtu_hbm, dti_hbm, decx_hbm, b0_hbm, b1_hbm,
              idx, urows, irows, b0b, b1b, pbuf0, pbuf1, s0, s1, s2):
    cid = lax.axis_index("c")
    sid = lax.axis_index("s")
    wid = sid * NCORE + cid
    lane = lax.broadcasted_iota(_i32, (LANES,), 0)
    zero16 = jnp.zeros((LANES,), _f32)
    sems = [s0, s1, s2]

    pltpu.sync_copy(decx_hbm.at[wid], idx)

    def _issue(slot, chunk):
        pltpu.async_copy(dtu_hbm.at[idx.at[0].at[chunk]], urows.at[slot],
                         sems[slot])
        pltpu.async_copy(dti_hbm.at[idx.at[1].at[chunk]], irows.at[slot],
                         sems[slot])

    def _drain(slot):
        pltpu.make_async_copy(dtu_hbm.at[pl.ds(0, SUB)], urows.at[slot],
                              sems[slot]).wait()
        pltpu.make_async_copy(dtu_hbm.at[pl.ds(0, SUB)], irows.at[slot],
                              sems[slot]).wait()

    for b in range(NBUF_E):
        _issue(b, b)

    def gk(g, c):
        for b in range(NBUF_E):
            _drain(b)

            def gp(gg, c2, b=b):
                e0 = gg * LANES
                for i in range(LANES):
                    e = e0 + i
                    uv = [urows[b, e, pl.ds(t * LANES, LANES)]
                          for t in range(8)]
                    iv = [irows[b, e, pl.ds(t * LANES, LANES)]
                          for t in range(4)]
                    p0 = uv[0] * iv[0] + uv[1] * iv[1] \
                        + uv[2] * iv[2] + uv[3] * iv[3]
                    p1 = uv[4] * iv[0] + uv[5] * iv[1] \
                        + uv[6] * iv[2] + uv[7] * iv[3]
                    pbuf0[i, pl.ds(0, LANES)] = p0
                    pbuf1[i, pl.ds(0, LANES)] = p1
                b0v = zero16
                b1v = zero16
                for t in range(LANES):
                    cc = jnp.full((LANES,), t, _i32)
                    b0v = b0v + plsc.load_gather(pbuf0, [lane, cc])
                    b1v = b1v + plsc.load_gather(pbuf1, [lane, cc])
                b0b[pl.ds(b * SUB + e0, LANES)] = b0v
                b1b[pl.ds(b * SUB + e0, LANES)] = b1v
                return c2

            lax.fori_loop(0, SUB // LANES, gp, 0)

            @pl.when(g < NGRP_E - 1)
            def _(b=b):
                _issue(b, (g + 1) * NBUF_E + b)

        base = wid * ED_T + g * NBUF_E * SUB
        pltpu.sync_copy(b0b, b0_hbm.at[pl.ds(base, NBUF_E * SUB)])
        pltpu.sync_copy(b1b, b1_hbm.at[pl.ds(base, NBUF_E * SUB)])
        return c

    lax.fori_loop(0, NGRP_E, gk, 0)


_dec_call = pl.kernel(
    _dec_body,
    out_type=[jax.ShapeDtypeStruct((EPAD_D,), _f32),
              jax.ShapeDtypeStruct((EPAD_D,), _f32)],
    mesh=plsc.VectorSubcoreMesh(core_axis_name="c", subcore_axis_name="s"),
    scratch_types=[pltpu.VMEM((2, NCH_E, SUB), _i32),
                   pltpu.VMEM((NBUF_E, SUB, 2 * OUT), _f32),
                   pltpu.VMEM((NBUF_E, SUB, 2 * OUT), _f32),
                   pltpu.VMEM((NBUF_E * SUB,), _f32),
                   pltpu.VMEM((NBUF_E * SUB,), _f32),
                   pltpu.VMEM((LANES, LANES + 1), _f32),
                   pltpu.VMEM((LANES, LANES + 1), _f32),
                   pltpu.SemaphoreType.DMA,
                   pltpu.SemaphoreType.DMA,
                   pltpu.SemaphoreType.DMA],
    compiler_params=pltpu.CompilerParams(needs_layout_passes=False),
)


BF = 3200


def _comb_body(b0, b1, cw, out):
    out[...] = b0[...] * cw[0:1, :] + b1[...] * cw[1:2, :]


def _comb_call(b0, b1, combine_W):
    return pl.pallas_call(
        _comb_body,
        grid=(E_DEC // BF,),
        in_specs=[
            pl.BlockSpec((BF, 1), lambda n: (n, 0)),
            pl.BlockSpec((BF, 1), lambda n: (n, 0)),
            pl.BlockSpec((NBASIS, R), lambda n: (0, 0)),
        ],
        out_specs=pl.BlockSpec((BF, R), lambda n: (n, 0)),
        out_shape=jax.ShapeDtypeStruct((E_DEC, R), _f32),
    )(b0, b1, combine_W)


def kernel(ufeat, ifeat, W_user, W_item, ufc_W, ufc_b, ifc_W, ifc_b, P,
           combine_W, enc_src, enc_dst, enc_etype, dec_u, dec_i):
    enc_src = enc_src.astype(_i32)
    enc_dst = enc_dst.astype(_i32)
    enc_etype = enc_etype.astype(_i32)
    dec_u = dec_u.astype(_i32)
    dec_i = dec_i.astype(_i32)

    degu, degi = _deg_call(enc_src, enc_dst)
    degu = degu.reshape(NSUB, NP)
    degi = degi.reshape(NSUB, NP)
    tu, ti = _tables_call(ufeat, ifeat, W_user, W_item, degu, degi)
    tu = tu.reshape(R * NU, AGG)
    ti = ti.reshape(R * NI, AGG)

    pc = EPAD_C - E_ENC
    src_p = jnp.concatenate([enc_src, jnp.full((pc,), NU, _i32)])
    dst_p = jnp.concatenate([enc_dst, jnp.full((pc,), NU, _i32)])
    et_p = jnp.concatenate([enc_etype, jnp.zeros((pc,), _i32)])
    encx = jnp.stack([src_p.reshape(-1, CH_CC), dst_p.reshape(-1, CH_CC),
                      et_p.reshape(-1, CH_CC)], axis=1)
    hi_raw, hu_raw = _enc_call(tu, ti, encx)

    dtu, dti = _fc_call(hi_raw[:NU], hu_raw[:NU], degu, degi,
                        ufc_W, ufc_b.reshape(1, OUT),
                        ifc_W, ifc_b.reshape(1, OUT), P)

    pd = EPAD_D - E_DEC
    du_p = jnp.concatenate([dec_u, jnp.zeros((pd,), _i32)])
    di_p = jnp.concatenate([dec_i, jnp.zeros((pd,), _i32)])
    decx = jnp.stack([du_p.reshape(32, NCH_E, SUB),
                      di_p.reshape(32, NCH_E, SUB)], axis=1)
    b0, b1 = _dec_call(dtu, dti, decx)

    return _comb_call(b0[:E_DEC].reshape(E_DEC, 1),
                      b1[:E_DEC].reshape(E_DEC, 1), combine_W)


# speedup vs baseline: 9.5032x; 1.0459x over previous
import jax
import jax.numpy as jnp
from jax import lax
from jax.experimental import pallas as pl
from jax.experimental.pallas import tpu as pltpu
from jax.experimental.pallas import tpu_sc as plsc

NU = 5000
NI = 5000
E_ENC = 320000
E_DEC = 320000
D_IN = 128
AGG = 128
OUT = 64
R = 5
NBASIS = 2

NCORE = 2
NSUB = 16
LANES = 16

NP = 5120
RPT = NP // NSUB

CH_A = 2000
EA_T = E_ENC // NSUB

SUB = 128
NS_C = 4
LOOK_C = 2
EC_T = 20480
EPAD_C = NSUB * EC_T
NCH_C = EC_T // SUB
NGRP_C = NCH_C // NS_C

ED_T = 10368
EPAD_D = 32 * ED_T
NCH_E = ED_T // SUB
NBUF_E = 3
NGRP_E = NCH_E // NBUF_E

_f32 = jnp.float32
_i32 = jnp.int32


def _deg_body(src_hbm, dst_hbm, degu_hbm, degi_hbm, ebuf, hist, red):
    cid = lax.axis_index("c")
    sid = lax.axis_index("s")
    lane = lax.broadcasted_iota(_i32, (LANES,), 0)
    ones = jnp.ones((LANES,), _f32)
    zero16 = jnp.zeros((LANES,), _f32)

    def zf(i, c):
        hist[pl.ds(i * LANES, LANES)] = zero16
        return c

    lax.fori_loop(0, (NSUB * NP) // LANES, zf, 0)

    def _process(edge_hbm):
        def ck(k, c):
            base = sid * EA_T + k * CH_A
            pltpu.sync_copy(edge_hbm.at[pl.ds(base, CH_A)], ebuf)

            def gp(g, c2):
                v = ebuf[pl.ds(g * LANES, LANES)]
                plsc.addupdate_scatter(hist, [lane * NP + v], ones)
                return c2

            lax.fori_loop(0, CH_A // LANES, gp, 0)
            return c

        lax.fori_loop(0, EA_T // CH_A, ck, 0)

        def rd(j, c):
            acc = zero16
            for l in range(LANES):
                acc = acc + hist[pl.ds(l * NP + j * LANES, LANES)]
            red[0, pl.ds(j * LANES, LANES)] = acc
            return c

        lax.fori_loop(0, NP // LANES, rd, 0)

    @pl.when(cid == 0)
    def _():
        _process(src_hbm)
        pltpu.sync_copy(red, degu_hbm.at[sid])

    @pl.when(cid == 1)
    def _():
        _process(dst_hbm)
        pltpu.sync_copy(red, degi_hbm.at[sid])


_deg_call = pl.kernel(
    _deg_body,
    out_type=[jax.ShapeDtypeStruct((NSUB, 1, NP), _f32),
              jax.ShapeDtypeStruct((NSUB, 1, NP), _f32)],
    mesh=plsc.VectorSubcoreMesh(core_axis_name="c", subcore_axis_name="s"),
    scratch_types=[pltpu.VMEM((CH_A,), _i32),
                   pltpu.VMEM((NSUB * NP,), _f32),
                   pltpu.VMEM((1, NP), _f32)],
    compiler_params=pltpu.CompilerParams(needs_layout_passes=False),
)


BN = 1000


def _tables_body(uf, vf, wu, wi, du, di, tu, ti):
    cu = lax.rsqrt(jnp.maximum(jnp.sum(du[...], axis=0)[:NU], 1.0))
    ci = lax.rsqrt(jnp.maximum(jnp.sum(di[...], axis=0)[:NI], 1.0))
    su = uf[...] * cu[:, None]
    si = vf[...] * ci[:, None]
    tu[...] = jnp.dot(su, wu[0], preferred_element_type=_f32)[None]
    ti[...] = jnp.dot(si, wi[0], preferred_element_type=_f32)[None]


def _tables_call(ufeat, ifeat, W_user, W_item, degu, degi):
    return pl.pallas_call(
        _tables_body,
        grid=(R,),
        in_specs=[
            pl.BlockSpec((NU, D_IN), lambda r: (0, 0)),
            pl.BlockSpec((NI, D_IN), lambda r: (0, 0)),
            pl.BlockSpec((1, D_IN, AGG), lambda r: (r, 0, 0)),
            pl.BlockSpec((1, D_IN, AGG), lambda r: (r, 0, 0)),
            pl.BlockSpec((NSUB, NP), lambda r: (0, 0)),
            pl.BlockSpec((NSUB, NP), lambda r: (0, 0)),
        ],
        out_specs=[
            pl.BlockSpec((1, NU, AGG), lambda r: (r, 0, 0)),
            pl.BlockSpec((1, NI, AGG), lambda r: (r, 0, 0)),
        ],
        out_shape=[jax.ShapeDtypeStruct((R, NU, AGG), _f32),
                   jax.ShapeDtypeStruct((R, NI, AGG), _f32)],
    )(ufeat, ifeat, W_user, W_item, degu, degi)


def _enc_body(tu_hbm, ti_hbm, encx_hbm, hi_hbm, hu_hbm,
              ib, kg, ks, rows, accum,
              gs0, gs1, gs2, gs3, ss0, ss1, ss2, ss3):
    cid = lax.axis_index("c")
    sid = lax.axis_index("s")
    zero16 = jnp.zeros((LANES,), _f32)
    gsems = [gs0, gs1, gs2, gs3]
    ssems = [ss0, ss1, ss2, ss3]

    def zf(i, c):
        for j in range(3):
            for t in range(SUB // LANES):
                rows[j, i, pl.ds(t * LANES, LANES)] = zero16
        return c

    lax.fori_loop(0, SUB, zf, 0)
    r0 = sid * RPT
    pltpu.sync_copy(rows.at[0], accum.at[pl.ds(r0, SUB)])
    pltpu.sync_copy(rows.at[1], accum.at[pl.ds(r0 + SUB, SUB)])
    pltpu.sync_copy(rows.at[2].at[pl.ds(0, RPT - 2 * SUB)],
                    accum.at[pl.ds(r0 + 2 * SUB, RPT - 2 * SUB)])
    plsc.subcore_barrier()

    def _run(table_hbm, g_row, s_row):
        def _keys(slot, chunk):
            pltpu.sync_copy(encx_hbm.at[sid].at[chunk], ib)

            def gp(i, c2):
                et = ib[2, pl.ds(i * LANES, LANES)]
                gv = ib[g_row, pl.ds(i * LANES, LANES)]
                sv = ib[s_row, pl.ds(i * LANES, LANES)]
                kg[slot, pl.ds(i * LANES, LANES)] = et * NU + gv
                ks[slot, pl.ds(i * LANES, LANES)] = sv
                return c2

            lax.fori_loop(0, SUB // LANES, gp, 0)

        def _gather(slot):
            pltpu.async_copy(table_hbm.at[kg.at[slot]], rows.at[slot],
                             gsems[slot])

        for b in range(LOOK_C):
            _keys(b, b)
            _gather(b)

        def ck(g, c):
            for b in range(NS_C):
                k = g * NS_C + b
                pltpu.make_async_copy(table_hbm.at[pl.ds(0, SUB)],
                                      rows.at[b], gsems[b]).wait()
                pltpu.async_copy(rows.at[b], accum.at[ks.at[b]], ssems[b],
                                 add=True)
                b2 = (b + LOOK_C) % NS_C

                @pl.when(k + LOOK_C < NCH_C)
                def _(b2=b2, k=k):
                    @pl.when(k >= NS_C - LOOK_C)
                    def _():
                        pltpu.make_async_copy(table_hbm.at[pl.ds(0, SUB)],
                                              rows.at[b2], ssems[b2]).wait()
                    _keys(b2, k + LOOK_C)
                    _gather(b2)
            return c

        lax.fori_loop(0, NGRP_C, ck, 0)
        for b in range(NS_C):
            pltpu.make_async_copy(table_hbm.at[pl.ds(0, SUB)],
                                  rows.at[b], ssems[b]).wait()

    @pl.when(cid == 0)
    def _():
        _run(tu_hbm, 0, 1)

    @pl.when(cid == 1)
    def _():
        _run(ti_hbm, 1, 0)

    plsc.subcore_barrier()

    @pl.when(cid == 0)
    def _():
        pltpu.sync_copy(accum.at[pl.ds(r0, RPT)], hi_hbm.at[pl.ds(r0, RPT)])

    @pl.when(cid == 1)
    def _():
        pltpu.sync_copy(accum.at[pl.ds(r0, RPT)], hu_hbm.at[pl.ds(r0, RPT)])


_enc_call = pl.kernel(
    _enc_body,
    out_type=[jax.ShapeDtypeStruct((NP, AGG), _f32),
              jax.ShapeDtypeStruct((NP, AGG), _f32)],
    mesh=plsc.VectorSubcoreMesh(core_axis_name="c", subcore_axis_name="s"),
    scratch_types=[pltpu.VMEM((3, SUB), _i32),
                   pltpu.VMEM((NS_C, SUB), _i32),
                   pltpu.VMEM((NS_C, SUB), _i32),
                   pltpu.VMEM((NS_C, SUB, AGG), _f32),
                   pltpu.VMEM_SHARED((NP, AGG), _f32),
                   pltpu.SemaphoreType.DMA,
                   pltpu.SemaphoreType.DMA,
                   pltpu.SemaphoreType.DMA,
                   pltpu.SemaphoreType.DMA,
                   pltpu.SemaphoreType.DMA,
                   pltpu.SemaphoreType.DMA,
                   pltpu.SemaphoreType.DMA,
                   pltpu.SemaphoreType.DMA],
    compiler_params=pltpu.CompilerParams(needs_layout_passes=False),
)


def _fc_body(hi, hu, du, di, ufw, ufb, ifw, ifb, p, dtu, dti):
    cu = lax.rsqrt(jnp.maximum(jnp.sum(du[...], axis=0)[:NU], 1.0))
    ci = lax.rsqrt(jnp.maximum(jnp.sum(di[...], axis=0)[:NI], 1.0))
    au = hu[...] * cu[:, None]
    ai = hi[...] * ci[:, None]
    au = jnp.where(au >= 0, au, 0.1 * au)
    ai = jnp.where(ai >= 0, ai, 0.1 * ai)
    uo = jnp.dot(au, ufw[...], preferred_element_type=_f32) + ufb[...]
    io = jnp.dot(ai, ifw[...], preferred_element_type=_f32) + ifb[...]
    dtu[...] = jnp.concatenate(
        [jnp.dot(uo, p[0], preferred_element_type=_f32),
         jnp.dot(uo, p[1], preferred_element_type=_f32)], axis=1)
    dti[...] = jnp.concatenate([io, jnp.zeros((NI, OUT), _f32)], axis=1)


def _fc_call(hi, hu, degu, degi, ufc_W, ufc_b, ifc_W, ifc_b, P):
    return pl.pallas_call(
        _fc_body,
        grid=(1,),
        in_specs=[
            pl.BlockSpec((NU, AGG), lambda n: (0, 0)),
            pl.BlockSpec((NU, AGG), lambda n: (0, 0)),
            pl.BlockSpec((NSUB, NP), lambda n: (0, 0)),
            pl.BlockSpec((NSUB, NP), lambda n: (0, 0)),
            pl.BlockSpec((AGG, OUT), lambda n: (0, 0)),
            pl.BlockSpec((1, OUT), lambda n: (0, 0)),
            pl.BlockSpec((AGG, OUT), lambda n: (0, 0)),
            pl.BlockSpec((1, OUT), lambda n: (0, 0)),
            pl.BlockSpec((NBASIS, OUT, OUT), lambda n: (0, 0, 0)),
        ],
        out_specs=[
            pl.BlockSpec((NU, 2 * OUT), lambda n: (0, 0)),
            pl.BlockSpec((NU, 2 * OUT), lambda n: (0, 0)),
        ],
        out_shape=[jax.ShapeDtypeStruct((NU, 2 * OUT), _f32),
                   jax.ShapeDtypeStruct((NI, 2 * OUT), _f32)],
    )(hi, hu, degu, degi, ufc_W, ufc_b, ifc_W, ifc_b, P)


def _dec_body(dtu_hbm, dti_hbm, decx_hbm, b0_hbm, b1_hbm,
              idx, urows, irows, b0b, b1b, pbuf0, pbuf1, s0, s1, s2):
    cid = lax.axis_index("c")
    sid = lax.axis_index("s")
    wid = sid * NCORE + cid
    lane = lax.broadcasted_iota(_i32, (LANES,), 0)
    zero16 = jnp.zeros((LANES,), _f32)
    sems = [s0, s1, s2]

    pltpu.sync_copy(decx_hbm.at[wid], idx)

    def _issue(slot, chunk):
        pltpu.async_copy(dtu_hbm.at[idx.at[0].at[chunk]], urows.at[slot],
                         sems[slot])
        pltpu.async_copy(dti_hbm.at[idx.at[1].at[chunk]], irows.at[slot],
                         sems[slot])

    def _drain(slot):
        pltpu.make_async_copy(dtu_hbm.at[pl.ds(0, SUB)], urows.at[slot],
                              sems[slot]).wait()
        pltpu.make_async_copy(dtu_hbm.at[pl.ds(0, SUB)], irows.at[slot],
                              sems[slot]).wait()

    for b in range(NBUF_E):
        _issue(b, b)

    def gk(g, c):
        for b in range(NBUF_E):
            _drain(b)

            def gp(gg, c2, b=b):
                e0 = gg * LANES
                for i in range(LANES):
                    e = e0 + i
                    uv = [urows[b, e, pl.ds(t * LANES, LANES)]
                          for t in range(8)]
                    iv = [irows[b, e, pl.ds(t * LANES, LANES)]
                          for t in range(4)]
                    p0 = uv[0] * iv[0] + uv[1] * iv[1] \
                        + uv[2] * iv[2] + uv[3] * iv[3]
                    p1 = uv[4] * iv[0] + uv[5] * iv[1] \
                        + uv[6] * iv[2] + uv[7] * iv[3]
                    pbuf0[i, pl.ds(0, LANES)] = p0
                    pbuf1[i, pl.ds(0, LANES)] = p1
                b0v = zero16
                b1v = zero16
                for t in range(LANES):
                    cc = jnp.full((LANES,), t, _i32)
                    b0v = b0v + plsc.load_gather(pbuf0, [lane, cc])
                    b1v = b1v + plsc.load_gather(pbuf1, [lane, cc])
                b0b[pl.ds(b * SUB + e0, LANES)] = b0v
                b1b[pl.ds(b * SUB + e0, LANES)] = b1v
                return c2

            lax.fori_loop(0, SUB // LANES, gp, 0)

            @pl.when(g < NGRP_E - 1)
            def _(b=b):
                _issue(b, (g + 1) * NBUF_E + b)

        base = wid * ED_T + g * NBUF_E * SUB
        pltpu.sync_copy(b0b, b0_hbm.at[pl.ds(base, NBUF_E * SUB)])
        pltpu.sync_copy(b1b, b1_hbm.at[pl.ds(base, NBUF_E * SUB)])
        return c

    lax.fori_loop(0, NGRP_E, gk, 0)


_dec_call = pl.kernel(
    _dec_body,
    out_type=[jax.ShapeDtypeStruct((EPAD_D,), _f32),
              jax.ShapeDtypeStruct((EPAD_D,), _f32)],
    mesh=plsc.VectorSubcoreMesh(core_axis_name="c", subcore_axis_name="s"),
    scratch_types=[pltpu.VMEM((2, NCH_E, SUB), _i32),
                   pltpu.VMEM((NBUF_E, SUB, 2 * OUT), _f32),
                   pltpu.VMEM((NBUF_E, SUB, 2 * OUT), _f32),
                   pltpu.VMEM((NBUF_E * SUB,), _f32),
                   pltpu.VMEM((NBUF_E * SUB,), _f32),
                   pltpu.VMEM((LANES, LANES + 1), _f32),
                   pltpu.VMEM((LANES, LANES + 1), _f32),
                   pltpu.SemaphoreType.DMA,
                   pltpu.SemaphoreType.DMA,
                   pltpu.SemaphoreType.DMA],
    compiler_params=pltpu.CompilerParams(needs_layout_passes=False),
)


BF = 3200


def _comb_body(b0, b1, cw, out):
    out[...] = b0[...] * cw[0:1, :] + b1[...] * cw[1:2, :]


def _comb_call(b0, b1, combine_W):
    return pl.pallas_call(
        _comb_body,
        grid=(E_DEC // BF,),
        in_specs=[
            pl.BlockSpec((BF, 1), lambda n: (n, 0)),
            pl.BlockSpec((BF, 1), lambda n: (n, 0)),
            pl.BlockSpec((NBASIS, R), lambda n: (0, 0)),
        ],
        out_specs=pl.BlockSpec((BF, R), lambda n: (n, 0)),
        out_shape=jax.ShapeDtypeStruct((E_DEC, R), _f32),
    )(b0, b1, combine_W)


def kernel(ufeat, ifeat, W_user, W_item, ufc_W, ufc_b, ifc_W, ifc_b, P,
           combine_W, enc_src, enc_dst, enc_etype, dec_u, dec_i):
    enc_src = enc_src.astype(_i32)
    enc_dst = enc_dst.astype(_i32)
    enc_etype = enc_etype.astype(_i32)
    dec_u = dec_u.astype(_i32)
    dec_i = dec_i.astype(_i32)

    degu, degi = _deg_call(enc_src, enc_dst)
    degu = degu.reshape(NSUB, NP)
    degi = degi.reshape(NSUB, NP)
    tu, ti = _tables_call(ufeat, ifeat, W_user, W_item, degu, degi)
    tu = tu.reshape(R * NU, AGG)
    ti = ti.reshape(R * NI, AGG)

    pc = EPAD_C - E_ENC
    src_p = jnp.concatenate([enc_src, jnp.full((pc,), NU, _i32)])
    dst_p = jnp.concatenate([enc_dst, jnp.full((pc,), NU, _i32)])
    et_p = jnp.concatenate([enc_etype, jnp.zeros((pc,), _i32)])
    encx = jnp.stack([src_p.reshape(NSUB, NCH_C, SUB),
                      dst_p.reshape(NSUB, NCH_C, SUB),
                      et_p.reshape(NSUB, NCH_C, SUB)], axis=2)
    hi_raw, hu_raw = _enc_call(tu, ti, encx)

    dtu, dti = _fc_call(hi_raw[:NU], hu_raw[:NU], degu, degi,
                        ufc_W, ufc_b.reshape(1, OUT),
                        ifc_W, ifc_b.reshape(1, OUT), P)

    pd = EPAD_D - E_DEC
    du_p = jnp.concatenate([dec_u, jnp.zeros((pd,), _i32)])
    di_p = jnp.concatenate([dec_i, jnp.zeros((pd,), _i32)])
    decx = jnp.stack([du_p.reshape(32, NCH_E, SUB),
                      di_p.reshape(32, NCH_E, SUB)], axis=1)
    b0, b1 = _dec_call(dtu, dti, decx)

    return _comb_call(b0[:E_DEC].reshape(E_DEC, 1),
                      b1[:E_DEC].reshape(E_DEC, 1), combine_W)


# speedup vs baseline: 11.8251x; 1.2443x over previous
import jax
import jax.numpy as jnp
from jax import lax
from jax.experimental import pallas as pl
from jax.experimental.pallas import tpu as pltpu
from jax.experimental.pallas import tpu_sc as plsc

NU = 5000
NI = 5000
E_ENC = 320000
E_DEC = 320000
D_IN = 128
AGG = 128
OUT = 64
R = 5
NBASIS = 2

NCORE = 2
NSUB = 16
LANES = 16

NP = 5120
RPT = NP // NSUB

CH_A = 2000
EA_T = E_ENC // NSUB

SUB = 128
NS_C = 4
LOOK_C = 2
EC_T = 20480
EPAD_C = NSUB * EC_T
NCH_C = EC_T // SUB
NGRP_C = NCH_C // NS_C

ED_T = 10368
EPAD_D = 32 * ED_T
NCH_E = ED_T // SUB
NBUF_E = 3
NGRP_E = NCH_E // NBUF_E

_f32 = jnp.float32
_i32 = jnp.int32


def _deg_body(src_hbm, dst_hbm, degu_hbm, degi_hbm, ebuf, hist, red):
    cid = lax.axis_index("c")
    sid = lax.axis_index("s")
    lane = lax.broadcasted_iota(_i32, (LANES,), 0)
    ones = jnp.ones((LANES,), _f32)
    zero16 = jnp.zeros((LANES,), _f32)

    def zf(i, c):
        hist[pl.ds(i * LANES, LANES)] = zero16
        return c

    lax.fori_loop(0, (NSUB * NP) // LANES, zf, 0)

    def _process(edge_hbm):
        def ck(k, c):
            base = sid * EA_T + k * CH_A
            pltpu.sync_copy(edge_hbm.at[pl.ds(base, CH_A)], ebuf)

            def gp(g, c2):
                v = ebuf[pl.ds(g * LANES, LANES)]
                plsc.addupdate_scatter(hist, [lane * NP + v], ones)
                return c2

            lax.fori_loop(0, CH_A // LANES, gp, 0)
            return c

        lax.fori_loop(0, EA_T // CH_A, ck, 0)

        def rd(j, c):
            acc = zero16
            for l in range(LANES):
                acc = acc + hist[pl.ds(l * NP + j * LANES, LANES)]
            red[0, pl.ds(j * LANES, LANES)] = acc
            return c

        lax.fori_loop(0, NP // LANES, rd, 0)

    @pl.when(cid == 0)
    def _():
        _process(src_hbm)
        pltpu.sync_copy(red, degu_hbm.at[sid])

    @pl.when(cid == 1)
    def _():
        _process(dst_hbm)
        pltpu.sync_copy(red, degi_hbm.at[sid])


_deg_call = pl.kernel(
    _deg_body,
    out_type=[jax.ShapeDtypeStruct((NSUB, 1, NP), _f32),
              jax.ShapeDtypeStruct((NSUB, 1, NP), _f32)],
    mesh=plsc.VectorSubcoreMesh(core_axis_name="c", subcore_axis_name="s"),
    scratch_types=[pltpu.VMEM((CH_A,), _i32),
                   pltpu.VMEM((NSUB * NP,), _f32),
                   pltpu.VMEM((1, NP), _f32)],
    compiler_params=pltpu.CompilerParams(needs_layout_passes=False),
)


BN = 1000


def _tables_body(uf, vf, wu, wi, du, di, tu, ti):
    cu = lax.rsqrt(jnp.maximum(jnp.sum(du[...], axis=0)[:NU], 1.0))
    ci = lax.rsqrt(jnp.maximum(jnp.sum(di[...], axis=0)[:NI], 1.0))
    su = uf[...] * cu[:, None]
    si = vf[...] * ci[:, None]
    tu[...] = jnp.dot(su, wu[0], preferred_element_type=_f32)[None]
    ti[...] = jnp.dot(si, wi[0], preferred_element_type=_f32)[None]


def _tables_call(ufeat, ifeat, W_user, W_item, degu, degi):
    return pl.pallas_call(
        _tables_body,
        grid=(R,),
        in_specs=[
            pl.BlockSpec((NU, D_IN), lambda r: (0, 0)),
            pl.BlockSpec((NI, D_IN), lambda r: (0, 0)),
            pl.BlockSpec((1, D_IN, AGG), lambda r: (r, 0, 0)),
            pl.BlockSpec((1, D_IN, AGG), lambda r: (r, 0, 0)),
            pl.BlockSpec((NSUB, NP), lambda r: (0, 0)),
            pl.BlockSpec((NSUB, NP), lambda r: (0, 0)),
        ],
        out_specs=[
            pl.BlockSpec((1, NU, AGG), lambda r: (r, 0, 0)),
            pl.BlockSpec((1, NI, AGG), lambda r: (r, 0, 0)),
        ],
        out_shape=[jax.ShapeDtypeStruct((R, NU, AGG), _f32),
                   jax.ShapeDtypeStruct((R, NI, AGG), _f32)],
    )(ufeat, ifeat, W_user, W_item, degu, degi)


def _enc_body(tu_hbm, ti_hbm, encx_hbm, hi_hbm, hu_hbm,
              ib, kg, ks, rows, accum,
              gs0, gs1, gs2, gs3, ss0, ss1, ss2, ss3):
    cid = lax.axis_index("c")
    sid = lax.axis_index("s")
    zero16 = jnp.zeros((LANES,), _f32)
    gsems = [gs0, gs1, gs2, gs3]
    ssems = [ss0, ss1, ss2, ss3]

    def zf(i, c):
        for j in range(3):
            for t in range(SUB // LANES):
                rows[j, i, pl.ds(t * LANES, LANES)] = zero16
        return c

    lax.fori_loop(0, SUB, zf, 0)
    r0 = sid * RPT
    pltpu.sync_copy(rows.at[0], accum.at[pl.ds(r0, SUB)])
    pltpu.sync_copy(rows.at[1], accum.at[pl.ds(r0 + SUB, SUB)])
    pltpu.sync_copy(rows.at[2].at[pl.ds(0, RPT - 2 * SUB)],
                    accum.at[pl.ds(r0 + 2 * SUB, RPT - 2 * SUB)])
    plsc.subcore_barrier()

    def _run(table_hbm, g_row, s_row):
        def _keys(slot, chunk):
            pltpu.sync_copy(encx_hbm.at[sid].at[chunk], ib)

            def gp(i, c2):
                et = ib[2, pl.ds(i * LANES, LANES)]
                gv = ib[g_row, pl.ds(i * LANES, LANES)]
                sv = ib[s_row, pl.ds(i * LANES, LANES)]
                kg[slot, pl.ds(i * LANES, LANES)] = et * NU + gv
                ks[slot, pl.ds(i * LANES, LANES)] = sv
                return c2

            lax.fori_loop(0, SUB // LANES, gp, 0)

        def _gather(slot):
            pltpu.async_copy(table_hbm.at[kg.at[slot]], rows.at[slot],
                             gsems[slot])

        for b in range(LOOK_C):
            _keys(b, b)
            _gather(b)

        def ck(g, c):
            for b in range(NS_C):
                k = g * NS_C + b
                pltpu.make_async_copy(table_hbm.at[pl.ds(0, SUB)],
                                      rows.at[b], gsems[b]).wait()
                pltpu.async_copy(rows.at[b], accum.at[ks.at[b]], ssems[b],
                                 add=True)
                b2 = (b + LOOK_C) % NS_C

                @pl.when(k + LOOK_C < NCH_C)
                def _(b2=b2, k=k):
                    @pl.when(k >= NS_C - LOOK_C)
                    def _():
                        pltpu.make_async_copy(table_hbm.at[pl.ds(0, SUB)],
                                              rows.at[b2], ssems[b2]).wait()
                    _keys(b2, k + LOOK_C)
                    _gather(b2)
            return c

        lax.fori_loop(0, NGRP_C, ck, 0)
        for b in range(NS_C):
            pltpu.make_async_copy(table_hbm.at[pl.ds(0, SUB)],
                                  rows.at[b], ssems[b]).wait()

    @pl.when(cid == 0)
    def _():
        _run(tu_hbm, 0, 1)

    @pl.when(cid == 1)
    def _():
        _run(ti_hbm, 1, 0)

    plsc.subcore_barrier()

    @pl.when(cid == 0)
    def _():
        pltpu.sync_copy(accum.at[pl.ds(r0, RPT)], hi_hbm.at[pl.ds(r0, RPT)])

    @pl.when(cid == 1)
    def _():
        pltpu.sync_copy(accum.at[pl.ds(r0, RPT)], hu_hbm.at[pl.ds(r0, RPT)])


_enc_call = pl.kernel(
    _enc_body,
    out_type=[jax.ShapeDtypeStruct((NP, AGG), _f32),
              jax.ShapeDtypeStruct((NP, AGG), _f32)],
    mesh=plsc.VectorSubcoreMesh(core_axis_name="c", subcore_axis_name="s"),
    scratch_types=[pltpu.VMEM((3, SUB), _i32),
                   pltpu.VMEM((NS_C, SUB), _i32),
                   pltpu.VMEM((NS_C, SUB), _i32),
                   pltpu.VMEM((NS_C, SUB, AGG), _f32),
                   pltpu.VMEM_SHARED((NP, AGG), _f32),
                   pltpu.SemaphoreType.DMA,
                   pltpu.SemaphoreType.DMA,
                   pltpu.SemaphoreType.DMA,
                   pltpu.SemaphoreType.DMA,
                   pltpu.SemaphoreType.DMA,
                   pltpu.SemaphoreType.DMA,
                   pltpu.SemaphoreType.DMA,
                   pltpu.SemaphoreType.DMA],
    compiler_params=pltpu.CompilerParams(needs_layout_passes=False),
)


def _fc_body(hi, hu, du, di, ufw, ufb, ifw, ifb, p, dtu, dti):
    cu = lax.rsqrt(jnp.maximum(jnp.sum(du[...], axis=0)[:NU], 1.0))
    ci = lax.rsqrt(jnp.maximum(jnp.sum(di[...], axis=0)[:NI], 1.0))
    au = hu[...] * cu[:, None]
    ai = hi[...] * ci[:, None]
    au = jnp.where(au >= 0, au, 0.1 * au)
    ai = jnp.where(ai >= 0, ai, 0.1 * ai)
    uo = jnp.dot(au, ufw[...], preferred_element_type=_f32) + ufb[...]
    io = jnp.dot(ai, ifw[...], preferred_element_type=_f32) + ifb[...]
    dtu[...] = jnp.concatenate(
        [jnp.dot(uo, p[0], preferred_element_type=_f32),
         jnp.dot(uo, p[1], preferred_element_type=_f32)], axis=1)
    dti[...] = jnp.concatenate([io, jnp.zeros((NI, OUT), _f32)], axis=1)


def _fc_call(hi, hu, degu, degi, ufc_W, ufc_b, ifc_W, ifc_b, P):
    return pl.pallas_call(
        _fc_body,
        grid=(1,),
        in_specs=[
            pl.BlockSpec((NU, AGG), lambda n: (0, 0)),
            pl.BlockSpec((NU, AGG), lambda n: (0, 0)),
            pl.BlockSpec((NSUB, NP), lambda n: (0, 0)),
            pl.BlockSpec((NSUB, NP), lambda n: (0, 0)),
            pl.BlockSpec((AGG, OUT), lambda n: (0, 0)),
            pl.BlockSpec((1, OUT), lambda n: (0, 0)),
            pl.BlockSpec((AGG, OUT), lambda n: (0, 0)),
            pl.BlockSpec((1, OUT), lambda n: (0, 0)),
            pl.BlockSpec((NBASIS, OUT, OUT), lambda n: (0, 0, 0)),
        ],
        out_specs=[
            pl.BlockSpec((NU, 2 * OUT), lambda n: (0, 0)),
            pl.BlockSpec((NI, 2 * OUT), lambda n: (0, 0)),
        ],
        out_shape=[jax.ShapeDtypeStruct((NU, 2 * OUT), _f32),
                   jax.ShapeDtypeStruct((NI, 2 * OUT), _f32)],
    )(hi, hu, degu, degi, ufc_W, ufc_b, ifc_W, ifc_b, P)


def _dec_body(dtu_hbm, dti_hbm, decx_hbm, b0_hbm, b1_hbm,
              idx, urows, irows, b0b, b1b, pbuf0, pbuf1,
              s0, s1, s2):
    cid = lax.axis_index("c")
    sid = lax.axis_index("s")
    wid = sid * NCORE + cid
    lane = lax.broadcasted_iota(_i32, (LANES,), 0)
    zero16 = jnp.zeros((LANES,), _f32)
    sems = [s0, s1, s2]

    pltpu.sync_copy(decx_hbm.at[wid], idx)

    def _issue(slot, chunk):
        pltpu.async_copy(dtu_hbm.at[idx.at[0].at[chunk]], urows.at[slot],
                         sems[slot])
        pltpu.async_copy(dti_hbm.at[idx.at[1].at[chunk]], irows.at[slot],
                         sems[slot])

    def _drain(slot):
        pltpu.make_async_copy(dtu_hbm.at[pl.ds(0, SUB)], urows.at[slot],
                              sems[slot]).wait()
        pltpu.make_async_copy(dti_hbm.at[pl.ds(0, SUB)], irows.at[slot],
                              sems[slot]).wait()

    for b in range(NBUF_E):
        _issue(b, b)

    def gk(g, c):
        for b in range(NBUF_E):
            _drain(b)

            def gp(gg, c2, b=b):
                e0 = gg * LANES
                for i in range(LANES):
                    e = e0 + i
                    uv = [urows[b, e, pl.ds(t * LANES, LANES)]
                          for t in range(8)]
                    iv = [irows[b, e, pl.ds(t * LANES, LANES)]
                          for t in range(4)]
                    p0 = uv[0] * iv[0] + uv[1] * iv[1] \
                        + uv[2] * iv[2] + uv[3] * iv[3]
                    p1 = uv[4] * iv[0] + uv[5] * iv[1] \
                        + uv[6] * iv[2] + uv[7] * iv[3]
                    pbuf0[i, pl.ds(0, LANES)] = p0
                    pbuf1[i, pl.ds(0, LANES)] = p1
                b0v = zero16
                b1v = zero16
                for t in range(LANES):
                    cc = jnp.full((LANES,), t, _i32)
                    b0v = b0v + plsc.load_gather(pbuf0, [lane, cc])
                    b1v = b1v + plsc.load_gather(pbuf1, [lane, cc])
                b0b[pl.ds(b * SUB + e0, LANES)] = b0v
                b1b[pl.ds(b * SUB + e0, LANES)] = b1v
                return c2

            lax.fori_loop(0, SUB // LANES, gp, 0)

            @pl.when(g < NGRP_E - 1)
            def _(b=b):
                _issue(b, (g + 1) * NBUF_E + b)

        base = wid * ED_T + g * NBUF_E * SUB
        pltpu.sync_copy(b0b, b0_hbm.at[pl.ds(base, NBUF_E * SUB)])
        pltpu.sync_copy(b1b, b1_hbm.at[pl.ds(base, NBUF_E * SUB)])
        return c

    lax.fori_loop(0, NGRP_E, gk, 0)


_dec_call = pl.kernel(
    _dec_body,
    out_type=[jax.ShapeDtypeStruct((EPAD_D,), _f32),
              jax.ShapeDtypeStruct((EPAD_D,), _f32)],
    mesh=plsc.VectorSubcoreMesh(core_axis_name="c", subcore_axis_name="s"),
    scratch_types=[pltpu.VMEM((2, NCH_E, SUB), _i32),
                   pltpu.VMEM((NBUF_E, SUB, 2 * OUT), _f32),
                   pltpu.VMEM((NBUF_E, SUB, 2 * OUT), _f32),
                   pltpu.VMEM((NBUF_E * SUB,), _f32),
                   pltpu.VMEM((NBUF_E * SUB,), _f32),
                   pltpu.VMEM((LANES, LANES + 1), _f32),
                   pltpu.VMEM((LANES, LANES + 1), _f32),
                   pltpu.SemaphoreType.DMA,
                   pltpu.SemaphoreType.DMA,
                   pltpu.SemaphoreType.DMA],
    compiler_params=pltpu.CompilerParams(needs_layout_passes=False),
)


RD = EPAD_D // 128


def _comb_body(b0, b1, cw, out):
    c = cw[...]
    a0 = b0[...]
    a1 = b1[...]
    for r in range(R):
        out[r] = a0 * c[0, r] + a1 * c[1, r]


def _comb_call(b0, b1, combine_W):
    return pl.pallas_call(
        _comb_body,
        grid=(1,),
        in_specs=[
            pl.BlockSpec((RD, 128), lambda n: (0, 0)),
            pl.BlockSpec((RD, 128), lambda n: (0, 0)),
            pl.BlockSpec((NBASIS, R), lambda n: (0, 0)),
        ],
        out_specs=pl.BlockSpec((R, RD, 128), lambda n: (0, 0, 0)),
        out_shape=jax.ShapeDtypeStruct((R, RD, 128), _f32),
    )(b0, b1, combine_W)


def kernel(ufeat, ifeat, W_user, W_item, ufc_W, ufc_b, ifc_W, ifc_b, P,
           combine_W, enc_src, enc_dst, enc_etype, dec_u, dec_i):
    enc_src = enc_src.astype(_i32)
    enc_dst = enc_dst.astype(_i32)
    enc_etype = enc_etype.astype(_i32)
    dec_u = dec_u.astype(_i32)
    dec_i = dec_i.astype(_i32)

    degu, degi = _deg_call(enc_src, enc_dst)
    degu = degu.reshape(NSUB, NP)
    degi = degi.reshape(NSUB, NP)
    tu, ti = _tables_call(ufeat, ifeat, W_user, W_item, degu, degi)
    tu = tu.reshape(R * NU, AGG)
    ti = ti.reshape(R * NI, AGG)

    pc = EPAD_C - E_ENC
    src_p = jnp.concatenate([enc_src, jnp.full((pc,), NU, _i32)])
    dst_p = jnp.concatenate([enc_dst, jnp.full((pc,), NU, _i32)])
    et_p = jnp.concatenate([enc_etype, jnp.zeros((pc,), _i32)])
    encx = jnp.stack([src_p.reshape(NSUB, NCH_C, SUB),
                      dst_p.reshape(NSUB, NCH_C, SUB),
                      et_p.reshape(NSUB, NCH_C, SUB)], axis=2)
    hi_raw, hu_raw = _enc_call(tu, ti, encx)

    dtu, dti = _fc_call(hi_raw[:NU], hu_raw[:NU], degu, degi,
                        ufc_W, ufc_b.reshape(1, OUT),
                        ifc_W, ifc_b.reshape(1, OUT), P)

    pd = EPAD_D - E_DEC
    du_p = jnp.concatenate([dec_u, jnp.zeros((pd,), _i32)])
    di_p = jnp.concatenate([dec_i, jnp.zeros((pd,), _i32)])
    decx = jnp.stack([du_p.reshape(32, NCH_E, SUB),
                      di_p.reshape(32, NCH_E, SUB)], axis=1)
    b0, b1 = _dec_call(dtu, dti, decx)

    outf = _comb_call(b0.reshape(RD, 128), b1.reshape(RD, 128), combine_W)
    return outf.reshape(R, EPAD_D)[:, :E_DEC].T
